# baseline (speedup 1.0000x reference)
import functools

import jax
import jax.numpy as jnp
from jax import lax
from jax.experimental import pallas as pl
from jax.experimental.pallas import tpu as pltpu
from jax.experimental.pallas import tpu_sc as plsc

N_NODES = 10000
N_PAD = 10240
D_IN = 75
HEADS1 = 10
OUT1 = 75
OUT2 = 128
N_GRAPHS = 64
N_EDGES = 160000
E_TOT = N_EDGES + N_NODES
E_PAD = 172032
CPAD = 80
NCHUNK = HEADS1
CH_W = CPAD

CB_W = 96
_PERM = [0] * CB_W
for _c2 in range(CB_W // 32):
    for _k in range(16):
        _PERM[32 * _c2 + 2 * _k] = 32 * _c2 + _k
        _PERM[32 * _c2 + 2 * _k + 1] = 32 * _c2 + 16 + _k

NB1 = 512
EB = E_PAD // 32
B = 128
NBAT = EB // B
NBUF = 3
NGRP = NBAT // NBUF
ET = E_PAD // 16
B5 = 128
NBAT5 = EB // B5
NBAT51 = ET // B5
NROWS_T = N_PAD // 16

_f32 = jnp.float32
_i32 = jnp.int32

_SC_PARAMS = pltpu.CompilerParams(
    use_tc_tiling_on_sc=False, needs_layout_passes=False)


def _k1_body(x_ref, w_ref, ms_ref, md_ref, *outs):
    h = jnp.dot(x_ref[...], w_ref[...], preferred_element_type=_f32)
    for c in range(NCHUNK):
        outs[c][...] = h[:, c * CB_W:(c + 1) * CB_W].astype(jnp.bfloat16)
    outs[NCHUNK][...] = jnp.dot(x_ref[...], ms_ref[...],
                                preferred_element_type=_f32)
    outs[NCHUNK + 1][...] = jnp.dot(x_ref[...], md_ref[...],
                                    preferred_element_type=_f32)


def _k1(xp, w1pb, msrc, mdst):
    nblk = N_PAD // NB1
    hs = [jax.ShapeDtypeStruct((N_PAD, CB_W), jnp.bfloat16)
          for _ in range(NCHUNK)]
    return pl.pallas_call(
        _k1_body,
        grid=(nblk,),
        in_specs=[
            pl.BlockSpec((NB1, 80), lambda i: (i, 0)),
            pl.BlockSpec((80, HEADS1 * CB_W), lambda i: (0, 0)),
            pl.BlockSpec((80, 16), lambda i: (0, 0)),
            pl.BlockSpec((80, 16), lambda i: (0, 0)),
        ],
        out_specs=[pl.BlockSpec((NB1, CB_W), lambda i: (i, 0))] * NCHUNK
        + [pl.BlockSpec((NB1, 16), lambda i: (i, 0))] * 2,
        out_shape=hs + [jax.ShapeDtypeStruct((N_PAD, 16), _f32)] * 2,
    )(xp, w1pb, msrc, mdst)


def _k2b_body(dp_ref, out_ref):
    out_ref[...] = dp_ref[0] + dp_ref[1] + 1e-16


def _k2b(denp):
    nblk = N_PAD // NB1
    return pl.pallas_call(
        _k2b_body,
        grid=(nblk,),
        in_specs=[pl.BlockSpec((2, NB1, 16), lambda i: (0, i, 0))],
        out_specs=pl.BlockSpec((NB1, 16), lambda i: (i, 0)),
        out_shape=jax.ShapeDtypeStruct((N_PAD, 16), _f32),
    )(denp)


def _k4_body(op_ref, w2_ref, b1_ref, a2_ref, h2_ref, a2t_ref):
    acc = jnp.zeros((NB1, OUT2), _f32)
    for c in range(NCHUNK):
        g = op_ref[0, c] + op_ref[1, c] + b1_ref[c]
        g = jnp.where(g > 0, g, jnp.exp(g) - 1.0)
        acc = acc + jnp.dot(g, w2_ref[c], preferred_element_type=_f32)
    h2_ref[...] = acc
    a2t_ref[...] = jnp.dot(acc, a2_ref[...], preferred_element_type=_f32)


def _k4(outp, w2p, b1p, att2cat):
    nblk = N_PAD // NB1
    return pl.pallas_call(
        _k4_body,
        grid=(nblk,),
        in_specs=[
            pl.BlockSpec((2, NCHUNK, NB1, CH_W), lambda i: (0, 0, i, 0)),
            pl.BlockSpec((NCHUNK, CH_W, OUT2), lambda i: (0, 0, 0)),
            pl.BlockSpec((NCHUNK, CH_W), lambda i: (0, 0)),
            pl.BlockSpec((OUT2, 8), lambda i: (0, 0)),
        ],
        out_specs=[
            pl.BlockSpec((NB1, OUT2), lambda i: (i, 0)),
            pl.BlockSpec((NB1, 8), lambda i: (i, 0)),
        ],
        out_shape=[
            jax.ShapeDtypeStruct((N_PAD, OUT2), _f32),
            jax.ShapeDtypeStruct((N_PAD, 8), _f32),
        ],
    )(outp, w2p, b1p, att2cat)


def _k6_body(o_ref, b2_ref, bid_ref, fcw_ref, fcb_ref, gmax_ref, res_ref):
    i = pl.program_id(0)
    nblk = pl.num_programs(0)
    h = o_ref[0] + o_ref[1] + b2_ref[...]
    h = jnp.maximum(h, 0.0)
    bid = bid_ref[0, 0, :]
    rows = []
    for g in range(N_GRAPHS):
        m = (bid == g).astype(_f32)
        rows.append(jnp.max(h * m[:, None], axis=0)[None, :])
    bmax = jnp.concatenate(rows, axis=0)
    gm = jnp.where(i == 0, bmax, jnp.maximum(gmax_ref[...], bmax))
    gmax_ref[...] = gm

    @pl.when(i == nblk - 1)
    def _():
        res = jnp.dot(gm, fcw_ref[...], preferred_element_type=_f32)
        res_ref[...] = jnp.maximum(res + fcb_ref[...], 0.0)


def _k6(out2p, b2, batchp, fc_w, fc_b):
    nblk = N_PAD // 128
    return pl.pallas_call(
        _k6_body,
        grid=(nblk,),
        in_specs=[
            pl.BlockSpec((2, 128, OUT2), lambda i: (0, i, 0)),
            pl.BlockSpec((1, OUT2), lambda i: (0, 0)),
            pl.BlockSpec((1, 1, 128), lambda i: (i, 0, 0)),
            pl.BlockSpec((OUT2, OUT2), lambda i: (0, 0)),
            pl.BlockSpec((1, OUT2), lambda i: (0, 0)),
        ],
        out_specs=[
            pl.BlockSpec((N_GRAPHS, OUT2), lambda i: (0, 0)),
            pl.BlockSpec((N_GRAPHS, OUT2), lambda i: (0, 0)),
        ],
        out_shape=[
            jax.ShapeDtypeStruct((N_GRAPHS, OUT2), _f32),
            jax.ShapeDtypeStruct((N_GRAPHS, OUT2), _f32),
        ],
    )(out2p, b2, batchp, fc_w, fc_b)


def _zero_rows(zbuf, width):
    def zrow(e, _):
        for j in range(width // 16):
            zbuf[e, pl.ds(j * 16, 16)] = jnp.zeros((16,), _f32)
        return ()
    lax.fori_loop(0, zbuf.shape[0], zrow, ())


def _k2_body(src_h, dst_h, as_h, ad_h, ex_h, denp_h,
             sidx0, sidx1, sidx2, didx0, didx1, didx2,
             arow0, arow1, arow2, drow0, drow1, drow2, exb, den_sh,
             gs0, gs1, gs2):
    cid = lax.axis_index("c")
    sid = lax.axis_index("s")
    sidxs = (sidx0, sidx1, sidx2)
    didxs = (didx0, didx1, didx2)
    arows = (arow0, arow1, arow2)
    drows = (drow0, drow1, drow2)
    gsems = (gs0, gs1, gs2)
    _zero_rows(exb, 16)
    for r in range(NROWS_T // B):
        pltpu.sync_copy(exb, den_sh.at[pl.ds(sid * NROWS_T + r * B, B)])
    plsc.subcore_barrier()

    base = cid * (E_PAD // 2) + sid * EB

    def fetch(b, p):
        off = base + b * B
        pltpu.sync_copy(src_h.at[pl.ds(off, B)], sidxs[p])
        pltpu.sync_copy(dst_h.at[pl.ds(off, B)], didxs[p])
        pltpu.async_copy(as_h.at[sidxs[p]], arows[p], gsems[p])
        pltpu.async_copy(ad_h.at[didxs[p]], drows[p], gsems[p])

    for p in range(NBUF - 1):
        fetch(p, p)

    def group(g, _):
        for p in range(NBUF):
            b = g * NBUF + p
            pltpu.make_async_copy(as_h.at[sidxs[p]], arows[p],
                                  gsems[p]).wait()
            pltpu.make_async_copy(ad_h.at[didxs[p]], drows[p],
                                  gsems[p]).wait()

            def edge(e, _):
                a = arows[p][e] + drows[p][e]
                a = jnp.where(a >= 0, a, 0.2 * a)
                exb[e] = jnp.exp(a)
                return ()
            lax.fori_loop(0, B, edge, (), unroll=4)
            off = base + b * B
            pltpu.sync_copy(exb, ex_h.at[pl.ds(off, B)])
            pltpu.sync_copy(exb, den_sh.at[didxs[p]], add=True)

            @pl.when(b + NBUF - 1 < NBAT)
            def _():
                fetch(b + NBUF - 1, (p + NBUF - 1) % NBUF)
        return ()

    lax.fori_loop(0, NGRP, group, ())
    plsc.subcore_barrier()
    pltpu.sync_copy(den_sh.at[pl.ds(sid * NROWS_T, NROWS_T)],
                    denp_h.at[cid].at[pl.ds(sid * NROWS_T, NROWS_T)])


def _k2(src, dst, asrc_t, adst_t):
    mesh = plsc.VectorSubcoreMesh(core_axis_name="c", subcore_axis_name="s")
    f = pl.kernel(
        _k2_body,
        out_type=[
            jax.ShapeDtypeStruct((E_PAD, 16), _f32),
            jax.ShapeDtypeStruct((2, N_PAD, 16), _f32),
        ],
        mesh=mesh,
        compiler_params=_SC_PARAMS,
        scratch_types=(
            [pltpu.VMEM((B,), _i32)] * 6
            + [pltpu.VMEM((B, 16), _f32)] * 6
            + [pltpu.VMEM((B, 16), _f32),
               pltpu.VMEM_SHARED((N_PAD, 16), _f32)]
            + [pltpu.SemaphoreType.DMA] * 3
        ),
    )
    return f(src, dst, asrc_t, adst_t)


def _k3_body(src_h, dst_h, ex_h, den_h, *refs):
    h1cs = refs[:NCHUNK]
    outp_h, w_h = refs[NCHUNK], refs[NCHUNK + 1]
    (sidxc, didxc, exr, denr, wr0, wr1, wr2, rb0, rb1, rb2,
     rows0, rows1, rows2,
     acc_sh, dsem, gs0, gs1, gs2, ws0, ws1, ws2, ss0, ss1, ss2) = \
        refs[NCHUNK + 2:]
    wrs = (wr0, wr1, wr2)
    rowsb = (rb0, rb1, rb2)
    rows = (rows0, rows1, rows2)
    gsems = (gs0, gs1, gs2)
    wsems = (ws0, ws1, ws2)
    ssems = (ss0, ss1, ss2)
    cid = lax.axis_index("c")
    sid = lax.axis_index("s")
    base = cid * (E_PAD // 2) + sid * EB

    def batcha(b, _):
        off = base + b * B
        pltpu.sync_copy(src_h.at[pl.ds(off, B)], sidxc.at[b])
        pltpu.sync_copy(dst_h.at[pl.ds(off, B)], didxc.at[b])
        pltpu.sync_copy(ex_h.at[pl.ds(off, B)], exr)
        pltpu.async_copy(den_h.at[didxc.at[b]], denr, dsem).wait()

        def edge(e, _):
            wr0[e] = exr[e] / denr[e]
            return ()
        lax.fori_loop(0, B, edge, (), unroll=4)
        pltpu.sync_copy(wr0, w_h.at[pl.ds(off, B)])
        return ()

    lax.fori_loop(0, NBAT, batcha, ())

    for chunk in range(NCHUNK):
        plsc.subcore_barrier()
        _zero_rows(rows0, CH_W)
        for r in range(NROWS_T // B):
            pltpu.sync_copy(
                rows0, acc_sh.at[pl.ds(sid * NROWS_T + r * B, B)])
        plsc.subcore_barrier()

        h1c = h1cs[chunk]

        def fetch(b, p):
            pltpu.async_copy(h1c.at[sidxc.at[b]], rowsb[p], gsems[p])
            pltpu.async_copy(w_h.at[pl.ds(base + b * B, B)], wrs[p],
                             wsems[p])

        for p in range(NBUF - 1):
            fetch(p, p)

        def group(g, _):
            for p in range(NBUF):
                b = g * NBUF + p
                pltpu.make_async_copy(h1c.at[sidxc.at[b]], rowsb[p],
                                      gsems[p]).wait()
                pltpu.make_async_copy(w_h.at[pl.ds(0, B)], wrs[p],
                                      wsems[p]).wait()

                def edge(e, _):
                    w = wrs[p][e][chunk]
                    for c2 in range(CB_W // 32):
                        u = rowsb[p][e, pl.ds(32 * c2, 32)]
                        ui = plsc.bitcast(u, _i32)
                        av = plsc.bitcast(lax.shift_left(ui, 16), _f32)
                        bv = plsc.bitcast(
                            jnp.bitwise_and(ui, jnp.int32(-65536)), _f32)
                        rows[p][e, pl.ds(32 * c2, 16)] = av * w
                        if 32 * c2 + 32 <= CH_W:
                            rows[p][e, pl.ds(32 * c2 + 16, 16)] = bv * w
                    return ()
                lax.fori_loop(0, B, edge, (), unroll=4)

                @pl.when(g < NGRP - 1)
                def _():
                    pltpu.async_copy(rows[p], acc_sh.at[didxc.at[b]],
                                     ssems[p], add=True)

                @pl.when(g == NGRP - 1)
                def _():
                    pltpu.sync_copy(rows[p], acc_sh.at[didxc.at[b]],
                                    add=True)

                pnext = (p + NBUF - 1) % NBUF

                @pl.when(b + NBUF - 1 < NBAT)
                def _():
                    @pl.when(b >= 1)
                    def _():
                        pltpu.make_async_copy(
                            rows[pnext], acc_sh.at[didxc.at[0]],
                            ssems[pnext]).wait()
                    fetch(b + NBUF - 1, pnext)
            return ()

        lax.fori_loop(0, NGRP, group, ())
        plsc.subcore_barrier()
        pltpu.sync_copy(
            acc_sh.at[pl.ds(sid * NROWS_T, NROWS_T)],
            outp_h.at[cid].at[chunk].at[pl.ds(sid * NROWS_T, NROWS_T)])


def _k3(src, dst, ex, den, h1cs):
    mesh = plsc.VectorSubcoreMesh(core_axis_name="c", subcore_axis_name="s")
    f = pl.kernel(
        _k3_body,
        out_type=[
            jax.ShapeDtypeStruct((2, NCHUNK, N_PAD, CH_W), _f32),
            jax.ShapeDtypeStruct((E_PAD, 16), _f32),
        ],
        mesh=mesh,
        compiler_params=_SC_PARAMS,
        scratch_types=(
            [pltpu.VMEM((NBAT, B), _i32)] * 2
            + [pltpu.VMEM((B, 16), _f32)] * 2
            + [pltpu.VMEM((B, 16), _f32)] * 3
            + [pltpu.VMEM((B, CB_W), jnp.bfloat16)] * 3
            + [pltpu.VMEM((B, CH_W), _f32)] * 3
            + [pltpu.VMEM_SHARED((N_PAD, CH_W), _f32)]
            + [pltpu.SemaphoreType.DMA] * 10
        ),
    )
    return f(src, dst, ex, den, *h1cs)


def _k5_body(src_h, dst_h, a2s_h, a2d_h, h2_h, out2p_h,
             a2sl, a2dl, sidx0, sidx1, didx0, didx1, ex2b, den2l, rows, zb1,
             den2_sh, acc2_sh, sem, is0, is1):
    cid = lax.axis_index("c")
    sid = lax.axis_index("s")
    sidxs = (sidx0, sidx1)
    didxs = (didx0, didx1)
    isems = (is0, is1)
    pltpu.sync_copy(a2s_h, a2sl)
    pltpu.sync_copy(a2d_h, a2dl)

    def zrow1(e, _):
        zb1[pl.ds(e * 16, 16)] = jnp.zeros((16,), _f32)
        return ()
    lax.fori_loop(0, NROWS_T // 16, zrow1, ())
    pltpu.sync_copy(zb1, den2_sh.at[pl.ds(sid * NROWS_T, NROWS_T)])
    _zero_rows(rows, OUT2)
    for r in range(NROWS_T // B5):
        pltpu.sync_copy(rows, acc2_sh.at[pl.ds(sid * NROWS_T + r * B5, B5)])
    plsc.subcore_barrier()

    def fetch_idx(base, b, p):
        off = base + b * B5
        pltpu.async_copy(src_h.at[pl.ds(off, B5)], sidxs[p], isems[p])
        pltpu.async_copy(dst_h.at[pl.ds(off, B5)], didxs[p], isems[p])

    def wait_idx(p):
        pltpu.make_async_copy(src_h.at[pl.ds(0, B5)], sidxs[p],
                              isems[p]).wait()
        pltpu.make_async_copy(dst_h.at[pl.ds(0, B5)], didxs[p],
                              isems[p]).wait()

    base1 = sid * ET
    fetch_idx(base1, 0, 0)

    def group1(g, _):
        for p in range(2):
            b = g * 2 + p
            wait_idx(p)
            for gg in range(B5 // 16):
                s16 = sidxs[p][pl.ds(gg * 16, 16)]
                d16 = didxs[p][pl.ds(gg * 16, 16)]
                av = plsc.load_gather(a2sl, [s16])
                dv = plsc.load_gather(a2dl, [d16])
                al = av + dv
                al = jnp.where(al >= 0, al, 0.2 * al)
                ex2b[pl.ds(gg * 16, 16)] = jnp.exp(al)

            @pl.when(b + 1 < NBAT51)
            def _():
                fetch_idx(base1, b + 1, 1 - p)
            pltpu.sync_copy(ex2b, den2_sh.at[didxs[p]], add=True)
        return ()

    lax.fori_loop(0, NBAT51 // 2, group1, ())
    plsc.subcore_barrier()
    pltpu.sync_copy(den2_sh, den2l)

    base2 = sid * ET + cid * EB
    fetch_idx(base2, 0, 0)

    def group2(g, _):
        for p in range(2):
            b = g * 2 + p
            wait_idx(p)
            pltpu.async_copy(h2_h.at[sidxs[p]], rows, sem).wait()

            @pl.when(b + 1 < NBAT5)
            def _():
                fetch_idx(base2, b + 1, 1 - p)

            def grp(gg, _):
                s16 = sidxs[p][pl.ds(gg * 16, 16)]
                d16 = didxs[p][pl.ds(gg * 16, 16)]
                av = plsc.load_gather(a2sl, [s16])
                dv = plsc.load_gather(a2dl, [d16])
                al = av + dv
                al = jnp.where(al >= 0, al, 0.2 * al)
                e16 = jnp.exp(al)
                den16 = plsc.load_gather(den2l, [d16]) + 1e-16
                w16 = e16 / den16
                for l in range(16):
                    w = w16[l]
                    r = gg * 16 + l
                    for j in range(OUT2 // 16):
                        rows[r, pl.ds(j * 16, 16)] = (
                            rows[r, pl.ds(j * 16, 16)] * w)
                return ()
            lax.fori_loop(0, B5 // 16, grp, ())
            pltpu.sync_copy(rows, acc2_sh.at[didxs[p]], add=True)
        return ()

    lax.fori_loop(0, NBAT5 // 2, group2, ())
    plsc.subcore_barrier()
    pltpu.sync_copy(acc2_sh.at[pl.ds(sid * NROWS_T, NROWS_T)],
                    out2p_h.at[cid].at[pl.ds(sid * NROWS_T, NROWS_T)])


def _k5(src, dst, a2s, a2d, h2):
    mesh = plsc.VectorSubcoreMesh(core_axis_name="c", subcore_axis_name="s")
    f = pl.kernel(
        _k5_body,
        out_type=jax.ShapeDtypeStruct((2, N_PAD, OUT2), _f32),
        mesh=mesh,
        compiler_params=_SC_PARAMS,
        scratch_types=[
            pltpu.VMEM((N_PAD,), _f32),
            pltpu.VMEM((N_PAD,), _f32),
            pltpu.VMEM((B5,), _i32),
            pltpu.VMEM((B5,), _i32),
            pltpu.VMEM((B5,), _i32),
            pltpu.VMEM((B5,), _i32),
            pltpu.VMEM((B5,), _f32),
            pltpu.VMEM((N_PAD,), _f32),
            pltpu.VMEM((B5, OUT2), _f32),
            pltpu.VMEM((NROWS_T,), _f32),
            pltpu.VMEM_SHARED((N_PAD,), _f32),
            pltpu.VMEM_SHARED((N_PAD, OUT2), _f32),
            pltpu.SemaphoreType.DMA,
            pltpu.SemaphoreType.DMA,
            pltpu.SemaphoreType.DMA,
        ],
    )
    return f(src, dst, a2s, a2d, h2)


def _prep(x, edge_index, batch, W1, att_src1, att_dst1, b1, W2,
          att_src2, att_dst2):
    xp = jnp.zeros((N_PAD, 80), _f32).at[:N_NODES, :D_IN].set(x)
    w1h = W1.reshape(D_IN, HEADS1, OUT1)
    w1p = jnp.zeros((80, HEADS1, CPAD), _f32).at[:D_IN, :, :OUT1].set(w1h)
    w1p = w1p.reshape(80, HEADS1 * CPAD)
    a_src = jnp.zeros((HEADS1, CPAD, 16), _f32)
    a_dst = jnp.zeros((HEADS1, CPAD, 16), _f32)
    for h in range(HEADS1):
        a_src = a_src.at[h, :OUT1, h].set(att_src1[h])
        a_dst = a_dst.at[h, :OUT1, h].set(att_dst1[h])
    msrc = w1p @ a_src.reshape(HEADS1 * CPAD, 16)
    mdst = w1p @ a_dst.reshape(HEADS1 * CPAD, 16)
    w1e = jnp.zeros((80, HEADS1, CB_W), _f32).at[:, :, :CPAD].set(
        w1p.reshape(80, HEADS1, CPAD))
    w1pb = w1e[:, :, jnp.array(_PERM)].reshape(80, HEADS1 * CB_W)
    w2h = W2.reshape(HEADS1, OUT1, OUT2)
    w2p = jnp.zeros((HEADS1, CPAD, OUT2), _f32).at[:, :OUT1, :].set(w2h)
    b1p = jnp.zeros((HEADS1, CPAD), _f32).at[:, :OUT1].set(
        b1.reshape(HEADS1, OUT1))
    att2cat = jnp.zeros((OUT2, 8), _f32)
    att2cat = att2cat.at[:, 0].set(att_src2[0]).at[:, 1].set(att_dst2[0])
    loop = jnp.arange(N_NODES, dtype=_i32)
    padv = jnp.full((E_PAD - E_TOT,), N_NODES, _i32)
    src = jnp.concatenate([edge_index[0], loop, padv])
    dst = jnp.concatenate([edge_index[1], loop, padv])
    batchp = jnp.concatenate(
        [batch, jnp.full((N_PAD - N_NODES,), N_GRAPHS, _i32)])
    return (xp, w1pb, msrc, mdst, w2p, b1p, att2cat, src, dst,
            batchp.reshape(N_PAD // 128, 1, 128))


def kernel(x, edge_index, batch, W1, att_src1, att_dst1, b1, W2, att_src2,
           att_dst2, b2, fc_w, fc_b):
    (xp, w1pb, msrc, mdst, w2p, b1p, att2cat, src, dst, batchp) = _prep(
        x, edge_index, batch, W1, att_src1, att_dst1, b1, W2,
        att_src2, att_dst2)
    outs = _k1(xp, w1pb, msrc, mdst)
    h1cs, asrc_t, adst_t = outs[:NCHUNK], outs[NCHUNK], outs[NCHUNK + 1]
    ex, denp = _k2(src, dst, asrc_t, adst_t)
    den = _k2b(denp)
    outp, _ = _k3(src, dst, ex, den, h1cs)
    h2, a2t = _k4(outp, w2p, b1p, att2cat)
    out2p = _k5(src, dst, a2t[:, 0], a2t[:, 1], h2)
    _, res = _k6(out2p, b2.reshape(1, OUT2), batchp, fc_w,
                 fc_b.reshape(1, OUT2))
    return res


# speedup vs baseline: 1.4513x; 1.4513x over previous
import functools

import jax
import jax.numpy as jnp
from jax import lax
from jax.experimental import pallas as pl
from jax.experimental.pallas import tpu as pltpu
from jax.experimental.pallas import tpu_sc as plsc

N_NODES = 10000
N_PAD = 10240
D_IN = 75
HEADS1 = 10
OUT1 = 75
OUT2 = 128
N_GRAPHS = 64
N_EDGES = 160000
E_TOT = N_EDGES + N_NODES
E_PAD = 172032
CPAD = 80
NCHUNK = HEADS1
CH_W = CPAD

CB_W = 96
_PERM = [0] * CB_W
for _c2 in range(CB_W // 32):
    for _k in range(16):
        _PERM[32 * _c2 + 2 * _k] = 32 * _c2 + _k
        _PERM[32 * _c2 + 2 * _k + 1] = 32 * _c2 + 16 + _k

NB1 = 512
EB = E_PAD // 32
B = 128
NBAT = EB // B
NBUF = 3
NGRP = NBAT // NBUF
ET = E_PAD // 16
B5 = 128
NBAT5 = EB // B5
NBAT51 = ET // B5
NROWS_T = N_PAD // 16

_f32 = jnp.float32
_i32 = jnp.int32

_SC_PARAMS = pltpu.CompilerParams(
    use_tc_tiling_on_sc=False, needs_layout_passes=False)


def _k1_body(x_ref, w_ref, ms_ref, md_ref, *outs):
    h = jnp.dot(x_ref[...], w_ref[...], preferred_element_type=_f32)
    for c in range(NCHUNK):
        outs[c][...] = h[:, c * CH_W:(c + 1) * CH_W]
    outs[NCHUNK][...] = jnp.dot(x_ref[...], ms_ref[...],
                                preferred_element_type=_f32)
    outs[NCHUNK + 1][...] = jnp.dot(x_ref[...], md_ref[...],
                                    preferred_element_type=_f32)


def _k1(xp, w1p, msrc, mdst):
    nblk = N_PAD // NB1
    hs = [jax.ShapeDtypeStruct((N_PAD, CH_W), _f32) for _ in range(NCHUNK)]
    return pl.pallas_call(
        _k1_body,
        grid=(nblk,),
        in_specs=[
            pl.BlockSpec((NB1, 80), lambda i: (i, 0)),
            pl.BlockSpec((80, HEADS1 * CPAD), lambda i: (0, 0)),
            pl.BlockSpec((80, 16), lambda i: (0, 0)),
            pl.BlockSpec((80, 16), lambda i: (0, 0)),
        ],
        out_specs=[pl.BlockSpec((NB1, CH_W), lambda i: (i, 0))] * NCHUNK
        + [pl.BlockSpec((NB1, 16), lambda i: (i, 0))] * 2,
        out_shape=hs + [jax.ShapeDtypeStruct((N_PAD, 16), _f32)] * 2,
    )(xp, w1p, msrc, mdst)


def _k2b_body(dp_ref, out_ref):
    out_ref[...] = dp_ref[0] + dp_ref[1] + 1e-16


def _k2b(denp):
    nblk = N_PAD // NB1
    return pl.pallas_call(
        _k2b_body,
        grid=(nblk,),
        in_specs=[pl.BlockSpec((2, NB1, 16), lambda i: (0, i, 0))],
        out_specs=pl.BlockSpec((NB1, 16), lambda i: (i, 0)),
        out_shape=jax.ShapeDtypeStruct((N_PAD, 16), _f32),
    )(denp)


def _k4_body(op_ref, w2_ref, b1_ref, a2_ref, h2_ref, a2t_ref):
    acc = jnp.zeros((NB1, OUT2), _f32)
    for c in range(NCHUNK):
        g = op_ref[0, c] + op_ref[1, c] + b1_ref[c]
        g = jnp.where(g > 0, g, jnp.exp(g) - 1.0)
        acc = acc + jnp.dot(g, w2_ref[c], preferred_element_type=_f32)
    h2_ref[...] = acc
    a2t_ref[...] = jnp.dot(acc, a2_ref[...], preferred_element_type=_f32)


def _k4(outp, w2p, b1p, att2cat):
    nblk = N_PAD // NB1
    return pl.pallas_call(
        _k4_body,
        grid=(nblk,),
        in_specs=[
            pl.BlockSpec((2, NCHUNK, NB1, CH_W), lambda i: (0, 0, i, 0)),
            pl.BlockSpec((NCHUNK, CH_W, OUT2), lambda i: (0, 0, 0)),
            pl.BlockSpec((NCHUNK, CH_W), lambda i: (0, 0)),
            pl.BlockSpec((OUT2, 8), lambda i: (0, 0)),
        ],
        out_specs=[
            pl.BlockSpec((NB1, OUT2), lambda i: (i, 0)),
            pl.BlockSpec((NB1, 8), lambda i: (i, 0)),
        ],
        out_shape=[
            jax.ShapeDtypeStruct((N_PAD, OUT2), _f32),
            jax.ShapeDtypeStruct((N_PAD, 8), _f32),
        ],
    )(outp, w2p, b1p, att2cat)


def _k6_body(o_ref, b2_ref, bid_ref, fcw_ref, fcb_ref, gmax_ref, res_ref):
    i = pl.program_id(0)
    nblk = pl.num_programs(0)
    h = o_ref[0] + o_ref[1] + b2_ref[...]
    h = jnp.maximum(h, 0.0)
    bid = bid_ref[0, 0, :]
    rows = []
    for g in range(N_GRAPHS):
        m = (bid == g).astype(_f32)
        rows.append(jnp.max(h * m[:, None], axis=0)[None, :])
    bmax = jnp.concatenate(rows, axis=0)
    gm = jnp.where(i == 0, bmax, jnp.maximum(gmax_ref[...], bmax))
    gmax_ref[...] = gm

    @pl.when(i == nblk - 1)
    def _():
        res = jnp.dot(gm, fcw_ref[...], preferred_element_type=_f32)
        res_ref[...] = jnp.maximum(res + fcb_ref[...], 0.0)


def _k6(out2p, b2, batchp, fc_w, fc_b):
    nblk = N_PAD // 128
    return pl.pallas_call(
        _k6_body,
        grid=(nblk,),
        in_specs=[
            pl.BlockSpec((2, 128, OUT2), lambda i: (0, i, 0)),
            pl.BlockSpec((1, OUT2), lambda i: (0, 0)),
            pl.BlockSpec((1, 1, 128), lambda i: (i, 0, 0)),
            pl.BlockSpec((OUT2, OUT2), lambda i: (0, 0)),
            pl.BlockSpec((1, OUT2), lambda i: (0, 0)),
        ],
        out_specs=[
            pl.BlockSpec((N_GRAPHS, OUT2), lambda i: (0, 0)),
            pl.BlockSpec((N_GRAPHS, OUT2), lambda i: (0, 0)),
        ],
        out_shape=[
            jax.ShapeDtypeStruct((N_GRAPHS, OUT2), _f32),
            jax.ShapeDtypeStruct((N_GRAPHS, OUT2), _f32),
        ],
    )(out2p, b2, batchp, fc_w, fc_b)


def _zero_rows(zbuf, width):
    def zrow(e, _):
        for j in range(width // 16):
            zbuf[e, pl.ds(j * 16, 16)] = jnp.zeros((16,), _f32)
        return ()
    lax.fori_loop(0, zbuf.shape[0], zrow, ())


def _k2_body(src_h, dst_h, as_h, ad_h, ex_h, denp_h,
             sidx0, sidx1, sidx2, didx0, didx1, didx2,
             arow0, arow1, arow2, drow0, drow1, drow2, exb, den_sh,
             gs0, gs1, gs2):
    cid = lax.axis_index("c")
    sid = lax.axis_index("s")
    sidxs = (sidx0, sidx1, sidx2)
    didxs = (didx0, didx1, didx2)
    arows = (arow0, arow1, arow2)
    drows = (drow0, drow1, drow2)
    gsems = (gs0, gs1, gs2)
    _zero_rows(exb, 16)
    for r in range(NROWS_T // B):
        pltpu.sync_copy(exb, den_sh.at[pl.ds(sid * NROWS_T + r * B, B)])
    plsc.subcore_barrier()

    base = cid * (E_PAD // 2) + sid * EB

    def fetch(b, p):
        off = base + b * B
        pltpu.sync_copy(src_h.at[pl.ds(off, B)], sidxs[p])
        pltpu.sync_copy(dst_h.at[pl.ds(off, B)], didxs[p])
        pltpu.async_copy(as_h.at[sidxs[p]], arows[p], gsems[p])
        pltpu.async_copy(ad_h.at[didxs[p]], drows[p], gsems[p])

    for p in range(NBUF - 1):
        fetch(p, p)

    def group(g, _):
        for p in range(NBUF):
            b = g * NBUF + p
            pltpu.make_async_copy(as_h.at[sidxs[p]], arows[p],
                                  gsems[p]).wait()
            pltpu.make_async_copy(ad_h.at[didxs[p]], drows[p],
                                  gsems[p]).wait()

            def edge(e, _):
                a = arows[p][e] + drows[p][e]
                a = jnp.where(a >= 0, a, 0.2 * a)
                exb[e] = jnp.exp(a)
                return ()
            lax.fori_loop(0, B, edge, (), unroll=4)
            off = base + b * B
            pltpu.sync_copy(exb, ex_h.at[pl.ds(off, B)])
            pltpu.sync_copy(exb, den_sh.at[didxs[p]], add=True)

            @pl.when(b + NBUF - 1 < NBAT)
            def _():
                fetch(b + NBUF - 1, (p + NBUF - 1) % NBUF)
        return ()

    lax.fori_loop(0, NGRP, group, ())
    plsc.subcore_barrier()
    pltpu.sync_copy(den_sh.at[pl.ds(sid * NROWS_T, NROWS_T)],
                    denp_h.at[cid].at[pl.ds(sid * NROWS_T, NROWS_T)])


def _k2(src, dst, asrc_t, adst_t):
    mesh = plsc.VectorSubcoreMesh(core_axis_name="c", subcore_axis_name="s")
    f = pl.kernel(
        _k2_body,
        out_type=[
            jax.ShapeDtypeStruct((E_PAD, 16), _f32),
            jax.ShapeDtypeStruct((2, N_PAD, 16), _f32),
        ],
        mesh=mesh,
        compiler_params=_SC_PARAMS,
        scratch_types=(
            [pltpu.VMEM((B,), _i32)] * 6
            + [pltpu.VMEM((B, 16), _f32)] * 6
            + [pltpu.VMEM((B, 16), _f32),
               pltpu.VMEM_SHARED((N_PAD, 16), _f32)]
            + [pltpu.SemaphoreType.DMA] * 3
        ),
    )
    return f(src, dst, asrc_t, adst_t)


def _k3_body(src_h, dst_h, ex_h, den_h, *refs):
    h1cs = refs[:NCHUNK]
    outp_h, w_h = refs[NCHUNK], refs[NCHUNK + 1]
    (sidxc, didxc, exr, denr, wr0, wr1, wr2, rows0, rows1, rows2,
     acc_sh, dsem, gs0, gs1, gs2, ws0, ws1, ws2, ss0, ss1, ss2) = \
        refs[NCHUNK + 2:]
    wrs = (wr0, wr1, wr2)
    rows = (rows0, rows1, rows2)
    gsems = (gs0, gs1, gs2)
    wsems = (ws0, ws1, ws2)
    ssems = (ss0, ss1, ss2)
    cid = lax.axis_index("c")
    sid = lax.axis_index("s")
    base = cid * (E_PAD // 2) + sid * EB

    def batcha(b, _):
        off = base + b * B
        pltpu.sync_copy(src_h.at[pl.ds(off, B)], sidxc.at[b])
        pltpu.sync_copy(dst_h.at[pl.ds(off, B)], didxc.at[b])
        pltpu.sync_copy(ex_h.at[pl.ds(off, B)], exr)
        pltpu.async_copy(den_h.at[didxc.at[b]], denr, dsem).wait()

        def edge(e, _):
            wr0[e] = exr[e] / denr[e]
            return ()
        lax.fori_loop(0, B, edge, (), unroll=4)
        pltpu.sync_copy(wr0, w_h.at[pl.ds(off, B)])
        return ()

    lax.fori_loop(0, NBAT, batcha, ())

    for chunk in range(NCHUNK):
        plsc.subcore_barrier()
        _zero_rows(rows0, CH_W)
        for r in range(NROWS_T // B):
            pltpu.sync_copy(
                rows0, acc_sh.at[pl.ds(sid * NROWS_T + r * B, B)])
        plsc.subcore_barrier()

        h1c = h1cs[chunk]

        def fetch(b, p):
            pltpu.async_copy(h1c.at[sidxc.at[b]], rows[p], gsems[p])
            pltpu.async_copy(w_h.at[pl.ds(base + b * B, B)], wrs[p],
                             wsems[p])

        for p in range(NBUF - 1):
            fetch(p, p)

        def group(g, _):
            for p in range(NBUF):
                b = g * NBUF + p
                pltpu.make_async_copy(h1c.at[sidxc.at[b]], rows[p],
                                      gsems[p]).wait()
                pltpu.make_async_copy(w_h.at[pl.ds(0, B)], wrs[p],
                                      wsems[p]).wait()

                def edge(e, _):
                    w = wrs[p][e][chunk]
                    for j in range(CH_W // 16):
                        rows[p][e, pl.ds(j * 16, 16)] = (
                            rows[p][e, pl.ds(j * 16, 16)] * w)
                    return ()
                lax.fori_loop(0, B, edge, (), unroll=4)

                @pl.when(g < NGRP - 1)
                def _():
                    pltpu.async_copy(rows[p], acc_sh.at[didxc.at[b]],
                                     ssems[p], add=True)

                @pl.when(g == NGRP - 1)
                def _():
                    pltpu.sync_copy(rows[p], acc_sh.at[didxc.at[b]],
                                    add=True)

                pnext = (p + NBUF - 1) % NBUF

                @pl.when(b + NBUF - 1 < NBAT)
                def _():
                    @pl.when(b >= 1)
                    def _():
                        pltpu.make_async_copy(
                            rows[pnext], acc_sh.at[didxc.at[0]],
                            ssems[pnext]).wait()
                    fetch(b + NBUF - 1, pnext)
            return ()

        lax.fori_loop(0, NGRP, group, ())
        plsc.subcore_barrier()
        pltpu.sync_copy(
            acc_sh.at[pl.ds(sid * NROWS_T, NROWS_T)],
            outp_h.at[cid].at[chunk].at[pl.ds(sid * NROWS_T, NROWS_T)])


def _k3(src, dst, ex, den, h1cs):
    mesh = plsc.VectorSubcoreMesh(core_axis_name="c", subcore_axis_name="s")
    f = pl.kernel(
        _k3_body,
        out_type=[
            jax.ShapeDtypeStruct((2, NCHUNK, N_PAD, CH_W), _f32),
            jax.ShapeDtypeStruct((E_PAD, 16), _f32),
        ],
        mesh=mesh,
        compiler_params=_SC_PARAMS,
        scratch_types=(
            [pltpu.VMEM((NBAT, B), _i32)] * 2
            + [pltpu.VMEM((B, 16), _f32)] * 2
            + [pltpu.VMEM((B, 16), _f32)] * 3
            + [pltpu.VMEM((B, CH_W), _f32)] * 3
            + [pltpu.VMEM_SHARED((N_PAD, CH_W), _f32)]
            + [pltpu.SemaphoreType.DMA] * 10
        ),
    )
    return f(src, dst, ex, den, *h1cs)


def _k5_body(src_h, dst_h, a2s_h, a2d_h, h2_h, out2p_h,
             a2sl, a2dl, sidx0, sidx1, didx0, didx1, ex2b, den2l, rows, zb1,
             den2_sh, acc2_sh, sem, is0, is1):
    cid = lax.axis_index("c")
    sid = lax.axis_index("s")
    sidxs = (sidx0, sidx1)
    didxs = (didx0, didx1)
    isems = (is0, is1)
    pltpu.sync_copy(a2s_h, a2sl)
    pltpu.sync_copy(a2d_h, a2dl)

    def zrow1(e, _):
        zb1[pl.ds(e * 16, 16)] = jnp.zeros((16,), _f32)
        return ()
    lax.fori_loop(0, NROWS_T // 16, zrow1, ())
    pltpu.sync_copy(zb1, den2_sh.at[pl.ds(sid * NROWS_T, NROWS_T)])
    _zero_rows(rows, OUT2)
    for r in range(NROWS_T // B5):
        pltpu.sync_copy(rows, acc2_sh.at[pl.ds(sid * NROWS_T + r * B5, B5)])
    plsc.subcore_barrier()

    def fetch_idx(base, b, p):
        off = base + b * B5
        pltpu.async_copy(src_h.at[pl.ds(off, B5)], sidxs[p], isems[p])
        pltpu.async_copy(dst_h.at[pl.ds(off, B5)], didxs[p], isems[p])

    def wait_idx(p):
        pltpu.make_async_copy(src_h.at[pl.ds(0, B5)], sidxs[p],
                              isems[p]).wait()
        pltpu.make_async_copy(dst_h.at[pl.ds(0, B5)], didxs[p],
                              isems[p]).wait()

    base1 = sid * ET
    fetch_idx(base1, 0, 0)

    def group1(g, _):
        for p in range(2):
            b = g * 2 + p
            wait_idx(p)
            for gg in range(B5 // 16):
                s16 = sidxs[p][pl.ds(gg * 16, 16)]
                d16 = didxs[p][pl.ds(gg * 16, 16)]
                av = plsc.load_gather(a2sl, [s16])
                dv = plsc.load_gather(a2dl, [d16])
                al = av + dv
                al = jnp.where(al >= 0, al, 0.2 * al)
                ex2b[pl.ds(gg * 16, 16)] = jnp.exp(al)

            @pl.when(b + 1 < NBAT51)
            def _():
                fetch_idx(base1, b + 1, 1 - p)
            pltpu.sync_copy(ex2b, den2_sh.at[didxs[p]], add=True)
        return ()

    lax.fori_loop(0, NBAT51 // 2, group1, ())
    plsc.subcore_barrier()
    pltpu.sync_copy(den2_sh, den2l)

    base2 = sid * ET + cid * EB
    fetch_idx(base2, 0, 0)

    def group2(g, _):
        for p in range(2):
            b = g * 2 + p
            wait_idx(p)
            pltpu.async_copy(h2_h.at[sidxs[p]], rows, sem).wait()

            @pl.when(b + 1 < NBAT5)
            def _():
                fetch_idx(base2, b + 1, 1 - p)

            def grp(gg, _):
                s16 = sidxs[p][pl.ds(gg * 16, 16)]
                d16 = didxs[p][pl.ds(gg * 16, 16)]
                av = plsc.load_gather(a2sl, [s16])
                dv = plsc.load_gather(a2dl, [d16])
                al = av + dv
                al = jnp.where(al >= 0, al, 0.2 * al)
                e16 = jnp.exp(al)
                den16 = plsc.load_gather(den2l, [d16]) + 1e-16
                w16 = e16 / den16
                for l in range(16):
                    w = w16[l]
                    r = gg * 16 + l
                    for j in range(OUT2 // 16):
                        rows[r, pl.ds(j * 16, 16)] = (
                            rows[r, pl.ds(j * 16, 16)] * w)
                return ()
            lax.fori_loop(0, B5 // 16, grp, ())
            pltpu.sync_copy(rows, acc2_sh.at[didxs[p]], add=True)
        return ()

    lax.fori_loop(0, NBAT5 // 2, group2, ())
    plsc.subcore_barrier()
    pltpu.sync_copy(acc2_sh.at[pl.ds(sid * NROWS_T, NROWS_T)],
                    out2p_h.at[cid].at[pl.ds(sid * NROWS_T, NROWS_T)])


def _k5(src, dst, a2s, a2d, h2):
    mesh = plsc.VectorSubcoreMesh(core_axis_name="c", subcore_axis_name="s")
    f = pl.kernel(
        _k5_body,
        out_type=jax.ShapeDtypeStruct((2, N_PAD, OUT2), _f32),
        mesh=mesh,
        compiler_params=_SC_PARAMS,
        scratch_types=[
            pltpu.VMEM((N_PAD,), _f32),
            pltpu.VMEM((N_PAD,), _f32),
            pltpu.VMEM((B5,), _i32),
            pltpu.VMEM((B5,), _i32),
            pltpu.VMEM((B5,), _i32),
            pltpu.VMEM((B5,), _i32),
            pltpu.VMEM((B5,), _f32),
            pltpu.VMEM((N_PAD,), _f32),
            pltpu.VMEM((B5, OUT2), _f32),
            pltpu.VMEM((NROWS_T,), _f32),
            pltpu.VMEM_SHARED((N_PAD,), _f32),
            pltpu.VMEM_SHARED((N_PAD, OUT2), _f32),
            pltpu.SemaphoreType.DMA,
            pltpu.SemaphoreType.DMA,
            pltpu.SemaphoreType.DMA,
        ],
    )
    return f(src, dst, a2s, a2d, h2)


def _prep(x, edge_index, batch, W1, att_src1, att_dst1, b1, W2,
          att_src2, att_dst2):
    xp = jnp.zeros((N_PAD, 80), _f32).at[:N_NODES, :D_IN].set(x)
    w1h = W1.reshape(D_IN, HEADS1, OUT1)
    w1p = jnp.zeros((80, HEADS1, CPAD), _f32).at[:D_IN, :, :OUT1].set(w1h)
    w1p = w1p.reshape(80, HEADS1 * CPAD)
    a_src = jnp.zeros((HEADS1, CPAD, 16), _f32)
    a_dst = jnp.zeros((HEADS1, CPAD, 16), _f32)
    for h in range(HEADS1):
        a_src = a_src.at[h, :OUT1, h].set(att_src1[h])
        a_dst = a_dst.at[h, :OUT1, h].set(att_dst1[h])
    msrc = w1p @ a_src.reshape(HEADS1 * CPAD, 16)
    mdst = w1p @ a_dst.reshape(HEADS1 * CPAD, 16)
    w2h = W2.reshape(HEADS1, OUT1, OUT2)
    w2p = jnp.zeros((HEADS1, CPAD, OUT2), _f32).at[:, :OUT1, :].set(w2h)
    b1p = jnp.zeros((HEADS1, CPAD), _f32).at[:, :OUT1].set(
        b1.reshape(HEADS1, OUT1))
    att2cat = jnp.zeros((OUT2, 8), _f32)
    att2cat = att2cat.at[:, 0].set(att_src2[0]).at[:, 1].set(att_dst2[0])
    loop = jnp.arange(N_NODES, dtype=_i32)
    padv = N_NODES + (jnp.arange(E_PAD - E_TOT, dtype=_i32)
                      % (N_PAD - N_NODES))
    src = jnp.concatenate([edge_index[0], loop, padv])
    dst = jnp.concatenate([edge_index[1], loop, padv])
    batchp = jnp.concatenate(
        [batch, jnp.full((N_PAD - N_NODES,), N_GRAPHS, _i32)])
    return (xp, w1p, msrc, mdst, w2p, b1p, att2cat, src, dst,
            batchp.reshape(N_PAD // 128, 1, 128))


def kernel(x, edge_index, batch, W1, att_src1, att_dst1, b1, W2, att_src2,
           att_dst2, b2, fc_w, fc_b):
    (xp, w1p, msrc, mdst, w2p, b1p, att2cat, src, dst, batchp) = _prep(
        x, edge_index, batch, W1, att_src1, att_dst1, b1, W2,
        att_src2, att_dst2)
    outs = _k1(xp, w1p, msrc, mdst)
    h1cs, asrc_t, adst_t = outs[:NCHUNK], outs[NCHUNK], outs[NCHUNK + 1]
    ex, denp = _k2(src, dst, asrc_t, adst_t)
    den = _k2b(denp)
    outp, _ = _k3(src, dst, ex, den, h1cs)
    h2, a2t = _k4(outp, w2p, b1p, att2cat)
    out2p = _k5(src, dst, a2t[:, 0], a2t[:, 1], h2)
    _, res = _k6(out2p, b2.reshape(1, OUT2), batchp, fc_w,
                 fc_b.reshape(1, OUT2))
    return res


# speedup vs baseline: 1.4990x; 1.0328x over previous
import functools

import jax
import jax.numpy as jnp
from jax import lax
from jax.experimental import pallas as pl
from jax.experimental.pallas import tpu as pltpu
from jax.experimental.pallas import tpu_sc as plsc

N_NODES = 10000
N_PAD = 10240
D_IN = 75
HEADS1 = 10
OUT1 = 75
OUT2 = 128
N_GRAPHS = 64
N_EDGES = 160000
E_TOT = N_EDGES + N_NODES
E_PAD = 172032
CPAD = 80
NCHUNK = HEADS1
CH_W = CPAD

CB_W = 96
_PERM = [0] * CB_W
for _c2 in range(CB_W // 32):
    for _k in range(16):
        _PERM[32 * _c2 + 2 * _k] = 32 * _c2 + _k
        _PERM[32 * _c2 + 2 * _k + 1] = 32 * _c2 + 16 + _k

NB1 = 512
EB = E_PAD // 32
B = 128
NBAT = EB // B
NBUF = 3
NGRP = NBAT // NBUF
ET = E_PAD // 16
B5 = 128
NBAT5 = EB // B5
NBAT51 = ET // B5
NROWS_T = N_PAD // 16

_f32 = jnp.float32
_i32 = jnp.int32

_SC_PARAMS = pltpu.CompilerParams(
    use_tc_tiling_on_sc=False, needs_layout_passes=False)


def _k1a_body(x_ref, ms_ref, md_ref, as_ref, ad_ref):
    as_ref[...] = jnp.dot(x_ref[...], ms_ref[...],
                          preferred_element_type=_f32)
    ad_ref[...] = jnp.dot(x_ref[...], md_ref[...],
                          preferred_element_type=_f32)


def _k1a(xp, msrc, mdst):
    nblk = N_PAD // NB1
    return pl.pallas_call(
        _k1a_body,
        grid=(nblk,),
        in_specs=[
            pl.BlockSpec((NB1, 80), lambda i: (i, 0)),
            pl.BlockSpec((80, 16), lambda i: (0, 0)),
            pl.BlockSpec((80, 16), lambda i: (0, 0)),
        ],
        out_specs=[pl.BlockSpec((NB1, 16), lambda i: (i, 0))] * 2,
        out_shape=[jax.ShapeDtypeStruct((N_PAD, 16), _f32)] * 2,
    )(xp, msrc, mdst)


def _k1_body(x_ref, w_ref, *outs):
    h = jnp.dot(x_ref[...], w_ref[...], preferred_element_type=_f32)
    for c in range(NCHUNK):
        outs[c][...] = h[:, c * CH_W:(c + 1) * CH_W]


def _k1(xp, w1p):
    nblk = N_PAD // NB1
    hs = [jax.ShapeDtypeStruct((N_PAD, CH_W), _f32) for _ in range(NCHUNK)]
    return pl.pallas_call(
        _k1_body,
        grid=(nblk,),
        in_specs=[
            pl.BlockSpec((NB1, 80), lambda i: (i, 0)),
            pl.BlockSpec((80, HEADS1 * CPAD), lambda i: (0, 0)),
        ],
        out_specs=[pl.BlockSpec((NB1, CH_W), lambda i: (i, 0))] * NCHUNK,
        out_shape=hs,
    )(xp, w1p)


def _k4_body(op_ref, w2_ref, b1_ref, a2_ref, h2_ref, a2t_ref):
    acc = jnp.zeros((NB1, OUT2), _f32)
    for c in range(NCHUNK):
        g = op_ref[0, c] + op_ref[1, c] + b1_ref[c]
        g = jnp.where(g > 0, g, jnp.exp(g) - 1.0)
        acc = acc + jnp.dot(g, w2_ref[c], preferred_element_type=_f32)
    h2_ref[...] = acc
    a2t_ref[...] = jnp.dot(acc, a2_ref[...], preferred_element_type=_f32)


def _k4(outp, w2p, b1p, att2cat):
    nblk = N_PAD // NB1
    return pl.pallas_call(
        _k4_body,
        grid=(nblk,),
        in_specs=[
            pl.BlockSpec((2, NCHUNK, NB1, CH_W), lambda i: (0, 0, i, 0)),
            pl.BlockSpec((NCHUNK, CH_W, OUT2), lambda i: (0, 0, 0)),
            pl.BlockSpec((NCHUNK, CH_W), lambda i: (0, 0)),
            pl.BlockSpec((OUT2, 8), lambda i: (0, 0)),
        ],
        out_specs=[
            pl.BlockSpec((NB1, OUT2), lambda i: (i, 0)),
            pl.BlockSpec((NB1, 8), lambda i: (i, 0)),
        ],
        out_shape=[
            jax.ShapeDtypeStruct((N_PAD, OUT2), _f32),
            jax.ShapeDtypeStruct((N_PAD, 8), _f32),
        ],
    )(outp, w2p, b1p, att2cat)


def _k6_body(o_ref, b2_ref, bid_ref, fcw_ref, fcb_ref, gmax_ref, res_ref):
    i = pl.program_id(0)
    nblk = pl.num_programs(0)
    h = o_ref[0] + o_ref[1] + b2_ref[...]
    h = jnp.maximum(h, 0.0)
    bid = bid_ref[0, 0, :]
    rows = []
    for g in range(N_GRAPHS):
        m = (bid == g).astype(_f32)
        rows.append(jnp.max(h * m[:, None], axis=0)[None, :])
    bmax = jnp.concatenate(rows, axis=0)
    gm = jnp.where(i == 0, bmax, jnp.maximum(gmax_ref[...], bmax))
    gmax_ref[...] = gm

    @pl.when(i == nblk - 1)
    def _():
        res = jnp.dot(gm, fcw_ref[...], preferred_element_type=_f32)
        res_ref[...] = jnp.maximum(res + fcb_ref[...], 0.0)


def _k6(out2p, b2, batchp, fc_w, fc_b):
    nblk = N_PAD // 128
    return pl.pallas_call(
        _k6_body,
        grid=(nblk,),
        in_specs=[
            pl.BlockSpec((2, 128, OUT2), lambda i: (0, i, 0)),
            pl.BlockSpec((1, OUT2), lambda i: (0, 0)),
            pl.BlockSpec((1, 1, 128), lambda i: (i, 0, 0)),
            pl.BlockSpec((OUT2, OUT2), lambda i: (0, 0)),
            pl.BlockSpec((1, OUT2), lambda i: (0, 0)),
        ],
        out_specs=[
            pl.BlockSpec((N_GRAPHS, OUT2), lambda i: (0, 0)),
            pl.BlockSpec((N_GRAPHS, OUT2), lambda i: (0, 0)),
        ],
        out_shape=[
            jax.ShapeDtypeStruct((N_GRAPHS, OUT2), _f32),
            jax.ShapeDtypeStruct((N_GRAPHS, OUT2), _f32),
        ],
    )(out2p, b2, batchp, fc_w, fc_b)


def _zero_rows(zbuf, width):
    def zrow(e, _):
        for j in range(width // 16):
            zbuf[e, pl.ds(j * 16, 16)] = jnp.zeros((16,), _f32)
        return ()
    lax.fori_loop(0, zbuf.shape[0], zrow, ())


def _k2_body(src_h, dst_h, as_h, ad_h, ex_h, denp_h,
             sidx0, sidx1, sidx2, didx0, didx1, didx2,
             arow0, arow1, arow2, drow0, drow1, drow2, exb, den_sh,
             gs0, gs1, gs2):
    cid = lax.axis_index("c")
    sid = lax.axis_index("s")
    sidxs = (sidx0, sidx1, sidx2)
    didxs = (didx0, didx1, didx2)
    arows = (arow0, arow1, arow2)
    drows = (drow0, drow1, drow2)
    gsems = (gs0, gs1, gs2)
    _zero_rows(exb, 16)
    for r in range(NROWS_T // B):
        pltpu.sync_copy(exb, den_sh.at[pl.ds(sid * NROWS_T + r * B, B)])
    plsc.subcore_barrier()

    base = cid * (E_PAD // 2) + sid * EB

    def fetch(b, p):
        off = base + b * B
        pltpu.sync_copy(src_h.at[pl.ds(off, B)], sidxs[p])
        pltpu.sync_copy(dst_h.at[pl.ds(off, B)], didxs[p])
        pltpu.async_copy(as_h.at[sidxs[p]], arows[p], gsems[p])
        pltpu.async_copy(ad_h.at[didxs[p]], drows[p], gsems[p])

    for p in range(NBUF - 1):
        fetch(p, p)

    def group(g, _):
        for p in range(NBUF):
            b = g * NBUF + p
            pltpu.make_async_copy(as_h.at[sidxs[p]], arows[p],
                                  gsems[p]).wait()
            pltpu.make_async_copy(ad_h.at[didxs[p]], drows[p],
                                  gsems[p]).wait()

            def edge(e, _):
                a = arows[p][e] + drows[p][e]
                a = jnp.where(a >= 0, a, 0.2 * a)
                exb[e] = jnp.exp(a)
                return ()
            lax.fori_loop(0, B, edge, (), unroll=4)
            off = base + b * B
            pltpu.sync_copy(exb, ex_h.at[pl.ds(off, B)])
            pltpu.sync_copy(exb, den_sh.at[didxs[p]], add=True)

            @pl.when(b + NBUF - 1 < NBAT)
            def _():
                fetch(b + NBUF - 1, (p + NBUF - 1) % NBUF)
        return ()

    lax.fori_loop(0, NGRP, group, ())
    plsc.subcore_barrier()
    pltpu.sync_copy(den_sh.at[pl.ds(sid * NROWS_T, NROWS_T)],
                    denp_h.at[cid].at[pl.ds(sid * NROWS_T, NROWS_T)])


def _k2(src, dst, asrc_t, adst_t):
    mesh = plsc.VectorSubcoreMesh(core_axis_name="c", subcore_axis_name="s")
    f = pl.kernel(
        _k2_body,
        out_type=[
            jax.ShapeDtypeStruct((E_PAD, 16), _f32),
            jax.ShapeDtypeStruct((2, N_PAD, 16), _f32),
        ],
        mesh=mesh,
        compiler_params=_SC_PARAMS,
        scratch_types=(
            [pltpu.VMEM((B,), _i32)] * 6
            + [pltpu.VMEM((B, 16), _f32)] * 6
            + [pltpu.VMEM((B, 16), _f32),
               pltpu.VMEM_SHARED((N_PAD, 16), _f32)]
            + [pltpu.SemaphoreType.DMA] * 3
        ),
    )
    return f(src, dst, asrc_t, adst_t)


def _k3_body(src_h, dst_h, ex_h, denp_h, *refs):
    h1cs = refs[:NCHUNK]
    outp_h, w_h = refs[NCHUNK], refs[NCHUNK + 1]
    (sidxc, didxc, exr, denr, wr0, wr1, wr2, rows0, rows1, rows2,
     acc_sh, den_sh, dsem, gs0, gs1, gs2, ws0, ws1, ws2, ss0, ss1, ss2) = \
        refs[NCHUNK + 2:]
    wrs = (wr0, wr1, wr2)
    rows = (rows0, rows1, rows2)
    gsems = (gs0, gs1, gs2)
    wsems = (ws0, ws1, ws2)
    ssems = (ss0, ss1, ss2)
    cid = lax.axis_index("c")
    sid = lax.axis_index("s")
    base = cid * (E_PAD // 2) + sid * EB

    for r in range(NROWS_T // B):
        roff = sid * NROWS_T + r * B
        pltpu.sync_copy(denp_h.at[0].at[pl.ds(roff, B)], exr)
        pltpu.sync_copy(denp_h.at[1].at[pl.ds(roff, B)], denr)

        def srow(e, _):
            wr0[e] = exr[e] + denr[e] + 1e-16
            return ()
        lax.fori_loop(0, B, srow, (), unroll=4)
        pltpu.sync_copy(wr0, den_sh.at[pl.ds(roff, B)])
    plsc.subcore_barrier()

    def batcha(b, _):
        off = base + b * B
        pltpu.sync_copy(src_h.at[pl.ds(off, B)], sidxc.at[b])
        pltpu.sync_copy(dst_h.at[pl.ds(off, B)], didxc.at[b])
        pltpu.sync_copy(ex_h.at[pl.ds(off, B)], exr)
        pltpu.async_copy(den_sh.at[didxc.at[b]], denr, dsem).wait()

        def edge(e, _):
            wr0[e] = exr[e] / denr[e]
            return ()
        lax.fori_loop(0, B, edge, (), unroll=4)
        pltpu.sync_copy(wr0, w_h.at[pl.ds(off, B)])
        return ()

    lax.fori_loop(0, NBAT, batcha, ())

    for chunk in range(NCHUNK):
        plsc.subcore_barrier()
        _zero_rows(rows0, CH_W)
        for r in range(NROWS_T // B):
            pltpu.sync_copy(
                rows0, acc_sh.at[pl.ds(sid * NROWS_T + r * B, B)])
        plsc.subcore_barrier()

        h1c = h1cs[chunk]

        def fetch(b, p):
            pltpu.async_copy(h1c.at[sidxc.at[b]], rows[p], gsems[p])
            pltpu.async_copy(w_h.at[pl.ds(base + b * B, B)], wrs[p],
                             wsems[p])

        for p in range(NBUF - 1):
            fetch(p, p)

        def group(g, _):
            for p in range(NBUF):
                b = g * NBUF + p
                pltpu.make_async_copy(h1c.at[sidxc.at[b]], rows[p],
                                      gsems[p]).wait()
                pltpu.make_async_copy(w_h.at[pl.ds(0, B)], wrs[p],
                                      wsems[p]).wait()

                def edge(e, _):
                    w = wrs[p][e][chunk]
                    for j in range(CH_W // 16):
                        rows[p][e, pl.ds(j * 16, 16)] = (
                            rows[p][e, pl.ds(j * 16, 16)] * w)
                    return ()
                lax.fori_loop(0, B, edge, (), unroll=4)

                @pl.when(g < NGRP - 1)
                def _():
                    pltpu.async_copy(rows[p], acc_sh.at[didxc.at[b]],
                                     ssems[p], add=True)

                @pl.when(g == NGRP - 1)
                def _():
                    pltpu.sync_copy(rows[p], acc_sh.at[didxc.at[b]],
                                    add=True)

                pnext = (p + NBUF - 1) % NBUF

                @pl.when(b + NBUF - 1 < NBAT)
                def _():
                    @pl.when(b >= 1)
                    def _():
                        pltpu.make_async_copy(
                            rows[pnext], acc_sh.at[didxc.at[0]],
                            ssems[pnext]).wait()
                    fetch(b + NBUF - 1, pnext)
            return ()

        lax.fori_loop(0, NGRP, group, ())
        plsc.subcore_barrier()
        pltpu.sync_copy(
            acc_sh.at[pl.ds(sid * NROWS_T, NROWS_T)],
            outp_h.at[cid].at[chunk].at[pl.ds(sid * NROWS_T, NROWS_T)])


def _k3(src, dst, ex, denp, h1cs):
    mesh = plsc.VectorSubcoreMesh(core_axis_name="c", subcore_axis_name="s")
    f = pl.kernel(
        _k3_body,
        out_type=[
            jax.ShapeDtypeStruct((2, NCHUNK, N_PAD, CH_W), _f32),
            jax.ShapeDtypeStruct((E_PAD, 16), _f32),
        ],
        mesh=mesh,
        compiler_params=_SC_PARAMS,
        scratch_types=(
            [pltpu.VMEM((NBAT, B), _i32)] * 2
            + [pltpu.VMEM((B, 16), _f32)] * 2
            + [pltpu.VMEM((B, 16), _f32)] * 3
            + [pltpu.VMEM((B, CH_W), _f32)] * 3
            + [pltpu.VMEM_SHARED((N_PAD, CH_W), _f32),
               pltpu.VMEM_SHARED((N_PAD, 16), _f32)]
            + [pltpu.SemaphoreType.DMA] * 10
        ),
    )
    return f(src, dst, ex, denp, *h1cs)


def _k5_body(src_h, dst_h, a2s_h, a2d_h, h2_h, out2p_h,
             a2sl, a2dl, sidx0, sidx1, didx0, didx1, ex2b, den2l, rows, zb1,
             den2_sh, acc2_sh, sem, is0, is1):
    cid = lax.axis_index("c")
    sid = lax.axis_index("s")
    sidxs = (sidx0, sidx1)
    didxs = (didx0, didx1)
    isems = (is0, is1)
    pltpu.sync_copy(a2s_h, a2sl)
    pltpu.sync_copy(a2d_h, a2dl)

    def zrow1(e, _):
        zb1[pl.ds(e * 16, 16)] = jnp.zeros((16,), _f32)
        return ()
    lax.fori_loop(0, NROWS_T // 16, zrow1, ())
    pltpu.sync_copy(zb1, den2_sh.at[pl.ds(sid * NROWS_T, NROWS_T)])
    _zero_rows(rows, OUT2)
    for r in range(NROWS_T // B5):
        pltpu.sync_copy(rows, acc2_sh.at[pl.ds(sid * NROWS_T + r * B5, B5)])
    plsc.subcore_barrier()

    def fetch_idx(base, b, p):
        off = base + b * B5
        pltpu.async_copy(src_h.at[pl.ds(off, B5)], sidxs[p], isems[p])
        pltpu.async_copy(dst_h.at[pl.ds(off, B5)], didxs[p], isems[p])

    def wait_idx(p):
        pltpu.make_async_copy(src_h.at[pl.ds(0, B5)], sidxs[p],
                              isems[p]).wait()
        pltpu.make_async_copy(dst_h.at[pl.ds(0, B5)], didxs[p],
                              isems[p]).wait()

    base1 = sid * ET
    fetch_idx(base1, 0, 0)

    def group1(g, _):
        for p in range(2):
            b = g * 2 + p
            wait_idx(p)
            for gg in range(B5 // 16):
                s16 = sidxs[p][pl.ds(gg * 16, 16)]
                d16 = didxs[p][pl.ds(gg * 16, 16)]
                av = plsc.load_gather(a2sl, [s16])
                dv = plsc.load_gather(a2dl, [d16])
                al = av + dv
                al = jnp.where(al >= 0, al, 0.2 * al)
                ex2b[pl.ds(gg * 16, 16)] = jnp.exp(al)

            @pl.when(b + 1 < NBAT51)
            def _():
                fetch_idx(base1, b + 1, 1 - p)
            pltpu.sync_copy(ex2b, den2_sh.at[didxs[p]], add=True)
        return ()

    lax.fori_loop(0, NBAT51 // 2, group1, ())
    plsc.subcore_barrier()
    pltpu.sync_copy(den2_sh, den2l)

    base2 = sid * ET + cid * EB
    fetch_idx(base2, 0, 0)

    def group2(g, _):
        for p in range(2):
            b = g * 2 + p
            wait_idx(p)
            pltpu.async_copy(h2_h.at[sidxs[p]], rows, sem).wait()

            @pl.when(b + 1 < NBAT5)
            def _():
                fetch_idx(base2, b + 1, 1 - p)

            def grp(gg, _):
                s16 = sidxs[p][pl.ds(gg * 16, 16)]
                d16 = didxs[p][pl.ds(gg * 16, 16)]
                av = plsc.load_gather(a2sl, [s16])
                dv = plsc.load_gather(a2dl, [d16])
                al = av + dv
                al = jnp.where(al >= 0, al, 0.2 * al)
                e16 = jnp.exp(al)
                den16 = plsc.load_gather(den2l, [d16]) + 1e-16
                w16 = e16 / den16
                for l in range(16):
                    w = w16[l]
                    r = gg * 16 + l
                    for j in range(OUT2 // 16):
                        rows[r, pl.ds(j * 16, 16)] = (
                            rows[r, pl.ds(j * 16, 16)] * w)
                return ()
            lax.fori_loop(0, B5 // 16, grp, ())
            pltpu.sync_copy(rows, acc2_sh.at[didxs[p]], add=True)
        return ()

    lax.fori_loop(0, NBAT5 // 2, group2, ())
    plsc.subcore_barrier()
    pltpu.sync_copy(acc2_sh.at[pl.ds(sid * NROWS_T, NROWS_T)],
                    out2p_h.at[cid].at[pl.ds(sid * NROWS_T, NROWS_T)])


def _k5(src, dst, a2s, a2d, h2):
    mesh = plsc.VectorSubcoreMesh(core_axis_name="c", subcore_axis_name="s")
    f = pl.kernel(
        _k5_body,
        out_type=jax.ShapeDtypeStruct((2, N_PAD, OUT2), _f32),
        mesh=mesh,
        compiler_params=_SC_PARAMS,
        scratch_types=[
            pltpu.VMEM((N_PAD,), _f32),
            pltpu.VMEM((N_PAD,), _f32),
            pltpu.VMEM((B5,), _i32),
            pltpu.VMEM((B5,), _i32),
            pltpu.VMEM((B5,), _i32),
            pltpu.VMEM((B5,), _i32),
            pltpu.VMEM((B5,), _f32),
            pltpu.VMEM((N_PAD,), _f32),
            pltpu.VMEM((B5, OUT2), _f32),
            pltpu.VMEM((NROWS_T,), _f32),
            pltpu.VMEM_SHARED((N_PAD,), _f32),
            pltpu.VMEM_SHARED((N_PAD, OUT2), _f32),
            pltpu.SemaphoreType.DMA,
            pltpu.SemaphoreType.DMA,
            pltpu.SemaphoreType.DMA,
        ],
    )
    return f(src, dst, a2s, a2d, h2)


def _prep(x, edge_index, batch, W1, att_src1, att_dst1, b1, W2,
          att_src2, att_dst2):
    xp = jnp.zeros((N_PAD, 80), _f32).at[:N_NODES, :D_IN].set(x)
    w1h = W1.reshape(D_IN, HEADS1, OUT1)
    w1p = jnp.zeros((80, HEADS1, CPAD), _f32).at[:D_IN, :, :OUT1].set(w1h)
    w1p = w1p.reshape(80, HEADS1 * CPAD)
    a_src = jnp.zeros((HEADS1, CPAD, 16), _f32)
    a_dst = jnp.zeros((HEADS1, CPAD, 16), _f32)
    for h in range(HEADS1):
        a_src = a_src.at[h, :OUT1, h].set(att_src1[h])
        a_dst = a_dst.at[h, :OUT1, h].set(att_dst1[h])
    msrc = w1p @ a_src.reshape(HEADS1 * CPAD, 16)
    mdst = w1p @ a_dst.reshape(HEADS1 * CPAD, 16)
    w2h = W2.reshape(HEADS1, OUT1, OUT2)
    w2p = jnp.zeros((HEADS1, CPAD, OUT2), _f32).at[:, :OUT1, :].set(w2h)
    b1p = jnp.zeros((HEADS1, CPAD), _f32).at[:, :OUT1].set(
        b1.reshape(HEADS1, OUT1))
    att2cat = jnp.zeros((OUT2, 8), _f32)
    att2cat = att2cat.at[:, 0].set(att_src2[0]).at[:, 1].set(att_dst2[0])
    loop = jnp.arange(N_NODES, dtype=_i32)
    padv = N_NODES + (jnp.arange(E_PAD - E_TOT, dtype=_i32)
                      % (N_PAD - N_NODES))
    src = jnp.concatenate([edge_index[0], loop, padv])
    dst = jnp.concatenate([edge_index[1], loop, padv])
    batchp = jnp.concatenate(
        [batch, jnp.full((N_PAD - N_NODES,), N_GRAPHS, _i32)])
    return (xp, w1p, msrc, mdst, w2p, b1p, att2cat, src, dst,
            batchp.reshape(N_PAD // 128, 1, 128))


def kernel(x, edge_index, batch, W1, att_src1, att_dst1, b1, W2, att_src2,
           att_dst2, b2, fc_w, fc_b):
    (xp, w1p, msrc, mdst, w2p, b1p, att2cat, src, dst, batchp) = _prep(
        x, edge_index, batch, W1, att_src1, att_dst1, b1, W2,
        att_src2, att_dst2)
    asrc_t, adst_t = _k1a(xp, msrc, mdst)
    h1cs = _k1(xp, w1p)
    ex, denp = _k2(src, dst, asrc_t, adst_t)
    outp, _ = _k3(src, dst, ex, denp, h1cs)
    h2, a2t = _k4(outp, w2p, b1p, att2cat)
    out2p = _k5(src, dst, a2t[:, 0], a2t[:, 1], h2)
    _, res = _k6(out2p, b2.reshape(1, OUT2), batchp, fc_w,
                 fc_b.reshape(1, OUT2))
    return res


# speedup vs baseline: 1.5187x; 1.0131x over previous
import functools

import jax
import jax.numpy as jnp
from jax import lax
from jax.experimental import pallas as pl
from jax.experimental.pallas import tpu as pltpu
from jax.experimental.pallas import tpu_sc as plsc

N_NODES = 10000
N_PAD = 10240
D_IN = 75
HEADS1 = 10
OUT1 = 75
OUT2 = 128
N_GRAPHS = 64
N_EDGES = 160000
E_TOT = N_EDGES + N_NODES
E_PAD = 172032
CPAD = 80
NCHUNK = HEADS1
CH_W = CPAD

CB_W = 96
_PERM = [0] * CB_W
for _c2 in range(CB_W // 32):
    for _k in range(16):
        _PERM[32 * _c2 + 2 * _k] = 32 * _c2 + _k
        _PERM[32 * _c2 + 2 * _k + 1] = 32 * _c2 + 16 + _k

NB1 = 512
EB = E_PAD // 32
B = 128
NBAT = EB // B
NBUF = 3
NGRP = NBAT // NBUF
ET = E_PAD // 16
B5 = 128
NBAT5 = EB // B5
NBAT51 = ET // B5
NROWS_T = N_PAD // 16

_f32 = jnp.float32
_i32 = jnp.int32

_SC_PARAMS = pltpu.CompilerParams(
    use_tc_tiling_on_sc=False, needs_layout_passes=False)


def _k1a_body(x_ref, ms_ref, md_ref, as_ref, ad_ref):
    as_ref[...] = jnp.dot(x_ref[...], ms_ref[...],
                          preferred_element_type=_f32)
    ad_ref[...] = jnp.dot(x_ref[...], md_ref[...],
                          preferred_element_type=_f32)


def _k1a(xp, msrc, mdst):
    nblk = N_PAD // NB1
    return pl.pallas_call(
        _k1a_body,
        grid=(nblk,),
        in_specs=[
            pl.BlockSpec((NB1, 80), lambda i: (i, 0)),
            pl.BlockSpec((80, 16), lambda i: (0, 0)),
            pl.BlockSpec((80, 16), lambda i: (0, 0)),
        ],
        out_specs=[pl.BlockSpec((NB1, 16), lambda i: (i, 0))] * 2,
        out_shape=[jax.ShapeDtypeStruct((N_PAD, 16), _f32)] * 2,
    )(xp, msrc, mdst)


def _k1_body(x_ref, w_ref, *outs):
    h = jnp.dot(x_ref[...], w_ref[...], preferred_element_type=_f32)
    for c in range(NCHUNK):
        outs[c][...] = h[:, c * CH_W:(c + 1) * CH_W]


def _k1(xp, w1p):
    nblk = N_PAD // NB1
    hs = [jax.ShapeDtypeStruct((N_PAD, CH_W), _f32) for _ in range(NCHUNK)]
    return pl.pallas_call(
        _k1_body,
        grid=(nblk,),
        in_specs=[
            pl.BlockSpec((NB1, 80), lambda i: (i, 0)),
            pl.BlockSpec((80, HEADS1 * CPAD), lambda i: (0, 0)),
        ],
        out_specs=[pl.BlockSpec((NB1, CH_W), lambda i: (i, 0))] * NCHUNK,
        out_shape=hs,
    )(xp, w1p)


def _k4_body(op_ref, w2_ref, b1_ref, a2_ref, h2_ref, a2t_ref):
    acc = jnp.zeros((NB1, OUT2), _f32)
    for c in range(NCHUNK):
        g = op_ref[0, c] + op_ref[1, c] + b1_ref[c]
        g = jnp.where(g > 0, g, jnp.exp(g) - 1.0)
        acc = acc + jnp.dot(g, w2_ref[c], preferred_element_type=_f32)
    h2_ref[...] = acc
    a2t_ref[...] = jnp.dot(acc, a2_ref[...], preferred_element_type=_f32)


def _k4(outp, w2p, b1p, att2cat):
    nblk = N_PAD // NB1
    return pl.pallas_call(
        _k4_body,
        grid=(nblk,),
        in_specs=[
            pl.BlockSpec((2, NCHUNK, NB1, CH_W), lambda i: (0, 0, i, 0)),
            pl.BlockSpec((NCHUNK, CH_W, OUT2), lambda i: (0, 0, 0)),
            pl.BlockSpec((NCHUNK, CH_W), lambda i: (0, 0)),
            pl.BlockSpec((OUT2, 8), lambda i: (0, 0)),
        ],
        out_specs=[
            pl.BlockSpec((NB1, OUT2), lambda i: (i, 0)),
            pl.BlockSpec((NB1, 8), lambda i: (i, 0)),
        ],
        out_shape=[
            jax.ShapeDtypeStruct((N_PAD, OUT2), _f32),
            jax.ShapeDtypeStruct((N_PAD, 8), _f32),
        ],
    )(outp, w2p, b1p, att2cat)


def _k6_body(o_ref, b2_ref, bid_ref, fcw_ref, fcb_ref, gmax_ref, res_ref):
    i = pl.program_id(0)
    nblk = pl.num_programs(0)
    h = o_ref[0] + o_ref[1] + b2_ref[...]
    h = jnp.maximum(h, 0.0)
    bid = bid_ref[0, 0, :]
    rows = []
    for g in range(N_GRAPHS):
        m = (bid == g).astype(_f32)
        rows.append(jnp.max(h * m[:, None], axis=0)[None, :])
    bmax = jnp.concatenate(rows, axis=0)
    gm = jnp.where(i == 0, bmax, jnp.maximum(gmax_ref[...], bmax))
    gmax_ref[...] = gm

    @pl.when(i == nblk - 1)
    def _():
        res = jnp.dot(gm, fcw_ref[...], preferred_element_type=_f32)
        res_ref[...] = jnp.maximum(res + fcb_ref[...], 0.0)


def _k6(out2p, b2, batchp, fc_w, fc_b):
    nblk = N_PAD // 128
    return pl.pallas_call(
        _k6_body,
        grid=(nblk,),
        in_specs=[
            pl.BlockSpec((2, 128, OUT2), lambda i: (0, i, 0)),
            pl.BlockSpec((1, OUT2), lambda i: (0, 0)),
            pl.BlockSpec((1, 1, 128), lambda i: (i, 0, 0)),
            pl.BlockSpec((OUT2, OUT2), lambda i: (0, 0)),
            pl.BlockSpec((1, OUT2), lambda i: (0, 0)),
        ],
        out_specs=[
            pl.BlockSpec((N_GRAPHS, OUT2), lambda i: (0, 0)),
            pl.BlockSpec((N_GRAPHS, OUT2), lambda i: (0, 0)),
        ],
        out_shape=[
            jax.ShapeDtypeStruct((N_GRAPHS, OUT2), _f32),
            jax.ShapeDtypeStruct((N_GRAPHS, OUT2), _f32),
        ],
    )(out2p, b2, batchp, fc_w, fc_b)


def _zero_rows(zbuf, width):
    def zrow(e, _):
        for j in range(width // 16):
            zbuf[e, pl.ds(j * 16, 16)] = jnp.zeros((16,), _f32)
        return ()
    lax.fori_loop(0, zbuf.shape[0], zrow, ())


def _k2_body(src_h, dst_h, as_h, ad_h, ex_h, denp_h,
             sidx0, sidx1, sidx2, didx0, didx1, didx2,
             arow0, arow1, arow2, drow0, drow1, drow2, exb, den_sh,
             gs0, gs1, gs2):
    cid = lax.axis_index("c")
    sid = lax.axis_index("s")
    sidxs = (sidx0, sidx1, sidx2)
    didxs = (didx0, didx1, didx2)
    arows = (arow0, arow1, arow2)
    drows = (drow0, drow1, drow2)
    gsems = (gs0, gs1, gs2)
    _zero_rows(exb, 16)
    for r in range(NROWS_T // B):
        pltpu.sync_copy(exb, den_sh.at[pl.ds(sid * NROWS_T + r * B, B)])
    plsc.subcore_barrier()

    base = cid * (E_PAD // 2) + sid * EB

    def fetch(b, p):
        off = base + b * B
        pltpu.sync_copy(src_h.at[pl.ds(off, B)], sidxs[p])
        pltpu.sync_copy(dst_h.at[pl.ds(off, B)], didxs[p])
        pltpu.async_copy(as_h.at[sidxs[p]], arows[p], gsems[p])
        pltpu.async_copy(ad_h.at[didxs[p]], drows[p], gsems[p])

    for p in range(NBUF - 1):
        fetch(p, p)

    def group(g, _):
        for p in range(NBUF):
            b = g * NBUF + p
            pltpu.make_async_copy(as_h.at[sidxs[p]], arows[p],
                                  gsems[p]).wait()
            pltpu.make_async_copy(ad_h.at[didxs[p]], drows[p],
                                  gsems[p]).wait()

            def edge(e, _):
                a = arows[p][e] + drows[p][e]
                a = jnp.where(a >= 0, a, 0.2 * a)
                exb[e] = jnp.exp(a)
                return ()
            lax.fori_loop(0, B, edge, (), unroll=4)
            off = base + b * B
            pltpu.sync_copy(exb, ex_h.at[pl.ds(off, B)])
            pltpu.sync_copy(exb, den_sh.at[didxs[p]], add=True)

            @pl.when(b + NBUF - 1 < NBAT)
            def _():
                fetch(b + NBUF - 1, (p + NBUF - 1) % NBUF)
        return ()

    lax.fori_loop(0, NGRP, group, ())
    plsc.subcore_barrier()
    pltpu.sync_copy(den_sh.at[pl.ds(sid * NROWS_T, NROWS_T)],
                    denp_h.at[cid].at[pl.ds(sid * NROWS_T, NROWS_T)])


def _k2(src, dst, asrc_t, adst_t):
    mesh = plsc.VectorSubcoreMesh(core_axis_name="c", subcore_axis_name="s")
    f = pl.kernel(
        _k2_body,
        out_type=[
            jax.ShapeDtypeStruct((E_PAD, 16), _f32),
            jax.ShapeDtypeStruct((2, N_PAD, 16), _f32),
        ],
        mesh=mesh,
        compiler_params=_SC_PARAMS,
        scratch_types=(
            [pltpu.VMEM((B,), _i32)] * 6
            + [pltpu.VMEM((B, 16), _f32)] * 6
            + [pltpu.VMEM((B, 16), _f32),
               pltpu.VMEM_SHARED((N_PAD, 16), _f32)]
            + [pltpu.SemaphoreType.DMA] * 3
        ),
    )
    return f(src, dst, asrc_t, adst_t)


def _k3_body(src_h, dst_h, ex_h, denp_h, *refs):
    h1cs = refs[:NCHUNK]
    outp_h, w_h = refs[NCHUNK], refs[NCHUNK + 1]
    (sidxc, didxc, exr, denr, wr0, wr1, wr2, rows0, rows1, rows2,
     acc_sh, den_sh, dsem, gs0, gs1, gs2, ws0, ws1, ws2, ss0, ss1, ss2) = \
        refs[NCHUNK + 2:]
    wrs = (wr0, wr1, wr2)
    rows = (rows0, rows1, rows2)
    gsems = (gs0, gs1, gs2)
    wsems = (ws0, ws1, ws2)
    ssems = (ss0, ss1, ss2)
    cid = lax.axis_index("c")
    sid = lax.axis_index("s")
    base = cid * (E_PAD // 2) + sid * EB

    for r in range(NROWS_T // B):
        roff = sid * NROWS_T + r * B
        pltpu.sync_copy(denp_h.at[0].at[pl.ds(roff, B)], exr)
        pltpu.sync_copy(denp_h.at[1].at[pl.ds(roff, B)], denr)

        def srow(e, _):
            wr0[e] = exr[e] + denr[e] + 1e-16
            return ()
        lax.fori_loop(0, B, srow, (), unroll=4)
        pltpu.sync_copy(wr0, den_sh.at[pl.ds(roff, B)])
    plsc.subcore_barrier()

    def batcha(b, _):
        off = base + b * B
        pltpu.sync_copy(dst_h.at[pl.ds(off, B)], didxc.at[b])
        cp = pltpu.async_copy(den_sh.at[didxc.at[b]], denr, dsem)
        pltpu.sync_copy(src_h.at[pl.ds(off, B)], sidxc.at[b])
        pltpu.sync_copy(ex_h.at[pl.ds(off, B)], exr)
        cp.wait()

        def edge(e, _):
            wr0[e] = exr[e] / denr[e]
            return ()
        lax.fori_loop(0, B, edge, (), unroll=4)
        pltpu.sync_copy(wr0, w_h.at[pl.ds(off, B)])
        return ()

    lax.fori_loop(0, NBAT, batcha, ())

    for chunk in range(NCHUNK):
        plsc.subcore_barrier()
        _zero_rows(rows0, CH_W)
        for r in range(NROWS_T // B):
            pltpu.sync_copy(
                rows0, acc_sh.at[pl.ds(sid * NROWS_T + r * B, B)])
        plsc.subcore_barrier()

        h1c = h1cs[chunk]

        def fetch(b, p):
            pltpu.async_copy(h1c.at[sidxc.at[b]], rows[p], gsems[p])
            pltpu.async_copy(w_h.at[pl.ds(base + b * B, B)], wrs[p],
                             wsems[p])

        for p in range(NBUF - 1):
            fetch(p, p)

        def group(g, _):
            for p in range(NBUF):
                b = g * NBUF + p
                pltpu.make_async_copy(h1c.at[sidxc.at[b]], rows[p],
                                      gsems[p]).wait()
                pltpu.make_async_copy(w_h.at[pl.ds(0, B)], wrs[p],
                                      wsems[p]).wait()

                def edge(e, _):
                    w = wrs[p][e][chunk]
                    for j in range(CH_W // 16):
                        rows[p][e, pl.ds(j * 16, 16)] = (
                            rows[p][e, pl.ds(j * 16, 16)] * w)
                    return ()
                lax.fori_loop(0, B, edge, (), unroll=4)

                @pl.when(g < NGRP - 1)
                def _():
                    pltpu.async_copy(rows[p], acc_sh.at[didxc.at[b]],
                                     ssems[p], add=True)

                @pl.when(g == NGRP - 1)
                def _():
                    pltpu.sync_copy(rows[p], acc_sh.at[didxc.at[b]],
                                    add=True)

                pnext = (p + NBUF - 1) % NBUF

                @pl.when(b + NBUF - 1 < NBAT)
                def _():
                    @pl.when(b >= 1)
                    def _():
                        pltpu.make_async_copy(
                            rows[pnext], acc_sh.at[didxc.at[0]],
                            ssems[pnext]).wait()
                    fetch(b + NBUF - 1, pnext)
            return ()

        lax.fori_loop(0, NGRP, group, ())
        plsc.subcore_barrier()
        pltpu.sync_copy(
            acc_sh.at[pl.ds(sid * NROWS_T, NROWS_T)],
            outp_h.at[cid].at[chunk].at[pl.ds(sid * NROWS_T, NROWS_T)])


def _k3(src, dst, ex, denp, h1cs):
    mesh = plsc.VectorSubcoreMesh(core_axis_name="c", subcore_axis_name="s")
    f = pl.kernel(
        _k3_body,
        out_type=[
            jax.ShapeDtypeStruct((2, NCHUNK, N_PAD, CH_W), _f32),
            jax.ShapeDtypeStruct((E_PAD, 16), _f32),
        ],
        mesh=mesh,
        compiler_params=_SC_PARAMS,
        scratch_types=(
            [pltpu.VMEM((NBAT, B), _i32)] * 2
            + [pltpu.VMEM((B, 16), _f32)] * 2
            + [pltpu.VMEM((B, 16), _f32)] * 3
            + [pltpu.VMEM((B, CH_W), _f32)] * 3
            + [pltpu.VMEM_SHARED((N_PAD, CH_W), _f32),
               pltpu.VMEM_SHARED((N_PAD, 16), _f32)]
            + [pltpu.SemaphoreType.DMA] * 10
        ),
    )
    return f(src, dst, ex, denp, *h1cs)


def _k5_body(src_h, dst_h, a2s_h, a2d_h, h2_h, out2p_h,
             a2sl, a2dl, sidx0, sidx1, didx0, didx1, ex2b, den2l, rows, zb1,
             den2_sh, acc2_sh, sem, is0, is1):
    cid = lax.axis_index("c")
    sid = lax.axis_index("s")
    sidxs = (sidx0, sidx1)
    didxs = (didx0, didx1)
    isems = (is0, is1)
    pltpu.sync_copy(a2s_h, a2sl)
    pltpu.sync_copy(a2d_h, a2dl)

    def zrow1(e, _):
        zb1[pl.ds(e * 16, 16)] = jnp.zeros((16,), _f32)
        return ()
    lax.fori_loop(0, NROWS_T // 16, zrow1, ())
    pltpu.sync_copy(zb1, den2_sh.at[pl.ds(sid * NROWS_T, NROWS_T)])
    _zero_rows(rows, OUT2)
    for r in range(NROWS_T // B5):
        pltpu.sync_copy(rows, acc2_sh.at[pl.ds(sid * NROWS_T + r * B5, B5)])
    plsc.subcore_barrier()

    def fetch_idx(base, b, p):
        off = base + b * B5
        pltpu.async_copy(src_h.at[pl.ds(off, B5)], sidxs[p], isems[p])
        pltpu.async_copy(dst_h.at[pl.ds(off, B5)], didxs[p], isems[p])

    def wait_idx(p):
        pltpu.make_async_copy(src_h.at[pl.ds(0, B5)], sidxs[p],
                              isems[p]).wait()
        pltpu.make_async_copy(dst_h.at[pl.ds(0, B5)], didxs[p],
                              isems[p]).wait()

    base1 = sid * ET
    fetch_idx(base1, 0, 0)

    def group1(g, _):
        for p in range(2):
            b = g * 2 + p
            wait_idx(p)

            @pl.when(b + 1 < NBAT51)
            def _():
                fetch_idx(base1, b + 1, 1 - p)
            for gg in range(B5 // 16):
                s16 = sidxs[p][pl.ds(gg * 16, 16)]
                d16 = didxs[p][pl.ds(gg * 16, 16)]
                av = plsc.load_gather(a2sl, [s16])
                dv = plsc.load_gather(a2dl, [d16])
                al = av + dv
                al = jnp.where(al >= 0, al, 0.2 * al)
                ex2b[pl.ds(gg * 16, 16)] = jnp.exp(al)
            pltpu.sync_copy(ex2b, den2_sh.at[didxs[p]], add=True)
        return ()

    lax.fori_loop(0, NBAT51 // 2, group1, ())
    plsc.subcore_barrier()
    pltpu.sync_copy(den2_sh, den2l)

    base2 = sid * ET + cid * EB
    fetch_idx(base2, 0, 0)

    def group2(g, _):
        for p in range(2):
            b = g * 2 + p
            wait_idx(p)

            @pl.when(b + 1 < NBAT5)
            def _():
                fetch_idx(base2, b + 1, 1 - p)
            pltpu.async_copy(h2_h.at[sidxs[p]], rows, sem).wait()

            def grp(gg, _):
                s16 = sidxs[p][pl.ds(gg * 16, 16)]
                d16 = didxs[p][pl.ds(gg * 16, 16)]
                av = plsc.load_gather(a2sl, [s16])
                dv = plsc.load_gather(a2dl, [d16])
                al = av + dv
                al = jnp.where(al >= 0, al, 0.2 * al)
                e16 = jnp.exp(al)
                den16 = plsc.load_gather(den2l, [d16]) + 1e-16
                w16 = e16 / den16
                for l in range(16):
                    w = w16[l]
                    r = gg * 16 + l
                    for j in range(OUT2 // 16):
                        rows[r, pl.ds(j * 16, 16)] = (
                            rows[r, pl.ds(j * 16, 16)] * w)
                return ()
            lax.fori_loop(0, B5 // 16, grp, ())
            pltpu.sync_copy(rows, acc2_sh.at[didxs[p]], add=True)
        return ()

    lax.fori_loop(0, NBAT5 // 2, group2, ())
    plsc.subcore_barrier()
    pltpu.sync_copy(acc2_sh.at[pl.ds(sid * NROWS_T, NROWS_T)],
                    out2p_h.at[cid].at[pl.ds(sid * NROWS_T, NROWS_T)])


def _k5(src, dst, a2s, a2d, h2):
    mesh = plsc.VectorSubcoreMesh(core_axis_name="c", subcore_axis_name="s")
    f = pl.kernel(
        _k5_body,
        out_type=jax.ShapeDtypeStruct((2, N_PAD, OUT2), _f32),
        mesh=mesh,
        compiler_params=_SC_PARAMS,
        scratch_types=[
            pltpu.VMEM((N_PAD,), _f32),
            pltpu.VMEM((N_PAD,), _f32),
            pltpu.VMEM((B5,), _i32),
            pltpu.VMEM((B5,), _i32),
            pltpu.VMEM((B5,), _i32),
            pltpu.VMEM((B5,), _i32),
            pltpu.VMEM((B5,), _f32),
            pltpu.VMEM((N_PAD,), _f32),
            pltpu.VMEM((B5, OUT2), _f32),
            pltpu.VMEM((NROWS_T,), _f32),
            pltpu.VMEM_SHARED((N_PAD,), _f32),
            pltpu.VMEM_SHARED((N_PAD, OUT2), _f32),
            pltpu.SemaphoreType.DMA,
            pltpu.SemaphoreType.DMA,
            pltpu.SemaphoreType.DMA,
        ],
    )
    return f(src, dst, a2s, a2d, h2)


def _prep(x, edge_index, batch, W1, att_src1, att_dst1, b1, W2,
          att_src2, att_dst2):
    xp = jnp.zeros((N_PAD, 80), _f32).at[:N_NODES, :D_IN].set(x)
    w1h = W1.reshape(D_IN, HEADS1, OUT1)
    w1p = jnp.zeros((80, HEADS1, CPAD), _f32).at[:D_IN, :, :OUT1].set(w1h)
    w1p = w1p.reshape(80, HEADS1 * CPAD)
    a_src = jnp.zeros((HEADS1, CPAD, 16), _f32)
    a_dst = jnp.zeros((HEADS1, CPAD, 16), _f32)
    for h in range(HEADS1):
        a_src = a_src.at[h, :OUT1, h].set(att_src1[h])
        a_dst = a_dst.at[h, :OUT1, h].set(att_dst1[h])
    msrc = w1p @ a_src.reshape(HEADS1 * CPAD, 16)
    mdst = w1p @ a_dst.reshape(HEADS1 * CPAD, 16)
    w2h = W2.reshape(HEADS1, OUT1, OUT2)
    w2p = jnp.zeros((HEADS1, CPAD, OUT2), _f32).at[:, :OUT1, :].set(w2h)
    b1p = jnp.zeros((HEADS1, CPAD), _f32).at[:, :OUT1].set(
        b1.reshape(HEADS1, OUT1))
    att2cat = jnp.zeros((OUT2, 8), _f32)
    att2cat = att2cat.at[:, 0].set(att_src2[0]).at[:, 1].set(att_dst2[0])
    loop = jnp.arange(N_NODES, dtype=_i32)
    padv = N_NODES + (jnp.arange(E_PAD - E_TOT, dtype=_i32)
                      % (N_PAD - N_NODES))
    src = jnp.concatenate([edge_index[0], loop, padv])
    dst = jnp.concatenate([edge_index[1], loop, padv])
    batchp = jnp.concatenate(
        [batch, jnp.full((N_PAD - N_NODES,), N_GRAPHS, _i32)])
    return (xp, w1p, msrc, mdst, w2p, b1p, att2cat, src, dst,
            batchp.reshape(N_PAD // 128, 1, 128))


def kernel(x, edge_index, batch, W1, att_src1, att_dst1, b1, W2, att_src2,
           att_dst2, b2, fc_w, fc_b):
    (xp, w1p, msrc, mdst, w2p, b1p, att2cat, src, dst, batchp) = _prep(
        x, edge_index, batch, W1, att_src1, att_dst1, b1, W2,
        att_src2, att_dst2)
    asrc_t, adst_t = _k1a(xp, msrc, mdst)
    h1cs = _k1(xp, w1p)
    ex, denp = _k2(src, dst, asrc_t, adst_t)
    outp, _ = _k3(src, dst, ex, denp, h1cs)
    h2, a2t = _k4(outp, w2p, b1p, att2cat)
    out2p = _k5(src, dst, a2t[:, 0], a2t[:, 1], h2)
    _, res = _k6(out2p, b2.reshape(1, OUT2), batchp, fc_w,
                 fc_b.reshape(1, OUT2))
    return res


# speedup vs baseline: 1.5542x; 1.0234x over previous
import functools

import jax
import jax.numpy as jnp
from jax import lax
from jax.experimental import pallas as pl
from jax.experimental.pallas import tpu as pltpu
from jax.experimental.pallas import tpu_sc as plsc

N_NODES = 10000
N_PAD = 10240
D_IN = 75
HEADS1 = 10
OUT1 = 75
OUT2 = 128
N_GRAPHS = 64
N_EDGES = 160000
E_TOT = N_EDGES + N_NODES
E_PAD = 172032
CPAD = 80
NCHUNK = HEADS1
CH_W = CPAD

CB_W = 96
_PERM = [0] * CB_W
for _c2 in range(CB_W // 32):
    for _k in range(16):
        _PERM[32 * _c2 + 2 * _k] = 32 * _c2 + _k
        _PERM[32 * _c2 + 2 * _k + 1] = 32 * _c2 + 16 + _k

NB1 = 512
EB = E_PAD // 32
B = 128
NBAT = EB // B
NBUF = 3
NGRP = NBAT // NBUF
ET = E_PAD // 16
B5 = 128
NBAT5 = EB // B5
NBAT51 = ET // B5
NROWS_T = N_PAD // 16

_f32 = jnp.float32
_i32 = jnp.int32

_SC_PARAMS = pltpu.CompilerParams(
    use_tc_tiling_on_sc=False, needs_layout_passes=False)


def _k1a_body(x_ref, ms_ref, md_ref, as_ref, ad_ref):
    as_ref[...] = jnp.dot(x_ref[...], ms_ref[...],
                          preferred_element_type=_f32)
    ad_ref[...] = jnp.dot(x_ref[...], md_ref[...],
                          preferred_element_type=_f32)


def _k1a(xp, msrc, mdst):
    nblk = N_PAD // NB1
    return pl.pallas_call(
        _k1a_body,
        grid=(nblk,),
        in_specs=[
            pl.BlockSpec((NB1, 80), lambda i: (i, 0)),
            pl.BlockSpec((80, 16), lambda i: (0, 0)),
            pl.BlockSpec((80, 16), lambda i: (0, 0)),
        ],
        out_specs=[pl.BlockSpec((NB1, 16), lambda i: (i, 0))] * 2,
        out_shape=[jax.ShapeDtypeStruct((N_PAD, 16), _f32)] * 2,
    )(xp, msrc, mdst)


def _k1_body(x_ref, w_ref, *outs):
    h = jnp.dot(x_ref[...], w_ref[...], preferred_element_type=_f32)
    for c in range(NCHUNK):
        outs[c][...] = h[:, c * CH_W:(c + 1) * CH_W]


def _k1(xp, w1p):
    nblk = N_PAD // NB1
    hs = [jax.ShapeDtypeStruct((N_PAD, CH_W), _f32) for _ in range(NCHUNK)]
    return pl.pallas_call(
        _k1_body,
        grid=(nblk,),
        in_specs=[
            pl.BlockSpec((NB1, 80), lambda i: (i, 0)),
            pl.BlockSpec((80, HEADS1 * CPAD), lambda i: (0, 0)),
        ],
        out_specs=[pl.BlockSpec((NB1, CH_W), lambda i: (i, 0))] * NCHUNK,
        out_shape=hs,
    )(xp, w1p)


def _k4_body(op_ref, w2_ref, b1_ref, a2_ref, h2_ref, a2t_ref):
    acc = jnp.zeros((NB1, OUT2), _f32)
    for c in range(NCHUNK):
        g = op_ref[0, c] + op_ref[1, c] + b1_ref[c]
        g = jnp.where(g > 0, g, jnp.exp(g) - 1.0)
        acc = acc + jnp.dot(g, w2_ref[c], preferred_element_type=_f32)
    h2_ref[...] = acc
    a2t_ref[...] = jnp.dot(acc, a2_ref[...], preferred_element_type=_f32)


def _k4(outp, w2p, b1p, att2cat):
    nblk = N_PAD // NB1
    return pl.pallas_call(
        _k4_body,
        grid=(nblk,),
        in_specs=[
            pl.BlockSpec((2, NCHUNK, NB1, CH_W), lambda i: (0, 0, i, 0)),
            pl.BlockSpec((NCHUNK, CH_W, OUT2), lambda i: (0, 0, 0)),
            pl.BlockSpec((NCHUNK, CH_W), lambda i: (0, 0)),
            pl.BlockSpec((OUT2, 8), lambda i: (0, 0)),
        ],
        out_specs=[
            pl.BlockSpec((NB1, OUT2), lambda i: (i, 0)),
            pl.BlockSpec((NB1, 8), lambda i: (i, 0)),
        ],
        out_shape=[
            jax.ShapeDtypeStruct((N_PAD, OUT2), _f32),
            jax.ShapeDtypeStruct((N_PAD, 8), _f32),
        ],
    )(outp, w2p, b1p, att2cat)


def _k6_body(o_ref, b2_ref, bid_ref, fcw_ref, fcb_ref, gmax_ref, res_ref):
    i = pl.program_id(0)
    nblk = pl.num_programs(0)
    h = o_ref[0] + o_ref[1] + b2_ref[...]
    h = jnp.maximum(h, 0.0)
    bid = bid_ref[0, 0, :]
    rows = []
    for g in range(N_GRAPHS):
        m = (bid == g).astype(_f32)
        rows.append(jnp.max(h * m[:, None], axis=0)[None, :])
    bmax = jnp.concatenate(rows, axis=0)
    gm = jnp.where(i == 0, bmax, jnp.maximum(gmax_ref[...], bmax))
    gmax_ref[...] = gm

    @pl.when(i == nblk - 1)
    def _():
        res = jnp.dot(gm, fcw_ref[...], preferred_element_type=_f32)
        res_ref[...] = jnp.maximum(res + fcb_ref[...], 0.0)


def _k6(out2p, b2, batchp, fc_w, fc_b):
    nblk = N_PAD // 128
    return pl.pallas_call(
        _k6_body,
        grid=(nblk,),
        in_specs=[
            pl.BlockSpec((2, 128, OUT2), lambda i: (0, i, 0)),
            pl.BlockSpec((1, OUT2), lambda i: (0, 0)),
            pl.BlockSpec((1, 1, 128), lambda i: (i, 0, 0)),
            pl.BlockSpec((OUT2, OUT2), lambda i: (0, 0)),
            pl.BlockSpec((1, OUT2), lambda i: (0, 0)),
        ],
        out_specs=[
            pl.BlockSpec((N_GRAPHS, OUT2), lambda i: (0, 0)),
            pl.BlockSpec((N_GRAPHS, OUT2), lambda i: (0, 0)),
        ],
        out_shape=[
            jax.ShapeDtypeStruct((N_GRAPHS, OUT2), _f32),
            jax.ShapeDtypeStruct((N_GRAPHS, OUT2), _f32),
        ],
    )(out2p, b2, batchp, fc_w, fc_b)


def _zero_rows(zbuf, width):
    def zrow(e, _):
        for j in range(width // 16):
            zbuf[e, pl.ds(j * 16, 16)] = jnp.zeros((16,), _f32)
        return ()
    lax.fori_loop(0, zbuf.shape[0], zrow, ())


def _k2_body(src_h, dst_h, as_h, ad_h, ex_h, denp_h,
             sidx0, sidx1, sidx2, didx0, didx1, didx2,
             arow0, arow1, arow2, drow0, drow1, drow2, exb, den_sh,
             gs0, gs1, gs2):
    cid = lax.axis_index("c")
    sid = lax.axis_index("s")
    sidxs = (sidx0, sidx1, sidx2)
    didxs = (didx0, didx1, didx2)
    arows = (arow0, arow1, arow2)
    drows = (drow0, drow1, drow2)
    gsems = (gs0, gs1, gs2)
    _zero_rows(exb, 16)
    for r in range(NROWS_T // B):
        pltpu.sync_copy(exb, den_sh.at[pl.ds(sid * NROWS_T + r * B, B)])
    plsc.subcore_barrier()

    base = cid * (E_PAD // 2) + sid * EB

    def fetch(b, p):
        off = base + b * B
        pltpu.sync_copy(src_h.at[pl.ds(off, B)], sidxs[p])
        pltpu.sync_copy(dst_h.at[pl.ds(off, B)], didxs[p])
        pltpu.async_copy(as_h.at[sidxs[p]], arows[p], gsems[p])
        pltpu.async_copy(ad_h.at[didxs[p]], drows[p], gsems[p])

    for p in range(NBUF - 1):
        fetch(p, p)

    def group(g, _):
        for p in range(NBUF):
            b = g * NBUF + p
            pltpu.make_async_copy(as_h.at[sidxs[p]], arows[p],
                                  gsems[p]).wait()
            pltpu.make_async_copy(ad_h.at[didxs[p]], drows[p],
                                  gsems[p]).wait()

            def edge(e, _):
                a = arows[p][e] + drows[p][e]
                a = jnp.where(a >= 0, a, 0.2 * a)
                exb[e] = jnp.exp(a)
                return ()
            lax.fori_loop(0, B, edge, (), unroll=4)
            off = base + b * B
            pltpu.sync_copy(exb, ex_h.at[pl.ds(off, B)])
            pltpu.sync_copy(exb, den_sh.at[didxs[p]], add=True)

            @pl.when(b + NBUF - 1 < NBAT)
            def _():
                fetch(b + NBUF - 1, (p + NBUF - 1) % NBUF)
        return ()

    lax.fori_loop(0, NGRP, group, ())
    plsc.subcore_barrier()
    pltpu.sync_copy(den_sh.at[pl.ds(sid * NROWS_T, NROWS_T)],
                    denp_h.at[cid].at[pl.ds(sid * NROWS_T, NROWS_T)])


def _k2(src, dst, asrc_t, adst_t):
    mesh = plsc.VectorSubcoreMesh(core_axis_name="c", subcore_axis_name="s")
    f = pl.kernel(
        _k2_body,
        out_type=[
            jax.ShapeDtypeStruct((E_PAD, 16), _f32),
            jax.ShapeDtypeStruct((2, N_PAD, 16), _f32),
        ],
        mesh=mesh,
        compiler_params=_SC_PARAMS,
        scratch_types=(
            [pltpu.VMEM((B,), _i32)] * 6
            + [pltpu.VMEM((B, 16), _f32)] * 6
            + [pltpu.VMEM((B, 16), _f32),
               pltpu.VMEM_SHARED((N_PAD, 16), _f32)]
            + [pltpu.SemaphoreType.DMA] * 3
        ),
    )
    return f(src, dst, asrc_t, adst_t)


def _k3_body(src_h, dst_h, ex_h, denp_h, *refs):
    h1cs = refs[:NCHUNK]
    outp_h, w_h = refs[NCHUNK], refs[NCHUNK + 1]
    (sidxc, didxc, exr, denr, exr2, denr2, wr0, wr1, wr2,
     rows0, rows1, rows2,
     acc_sh, den_sh, dsem, dsem2, es0, es1,
     gs0, gs1, gs2, ws0, ws1, ws2, ss0, ss1, ss2) = refs[NCHUNK + 2:]
    wrs = (wr0, wr1, wr2)
    rows = (rows0, rows1, rows2)
    gsems = (gs0, gs1, gs2)
    wsems = (ws0, ws1, ws2)
    ssems = (ss0, ss1, ss2)
    cid = lax.axis_index("c")
    sid = lax.axis_index("s")
    base = cid * (E_PAD // 2) + sid * EB

    for r in range(NROWS_T // B):
        roff = sid * NROWS_T + r * B
        pltpu.sync_copy(denp_h.at[0].at[pl.ds(roff, B)], exr)
        pltpu.sync_copy(denp_h.at[1].at[pl.ds(roff, B)], denr)

        def srow(e, _):
            wr0[e] = exr[e] + denr[e] + 1e-16
            return ()
        lax.fori_loop(0, B, srow, (), unroll=4)
        pltpu.sync_copy(wr0, den_sh.at[pl.ds(roff, B)])
    plsc.subcore_barrier()

    exrs = (exr, exr2)
    denrs = (denr, denr2)
    dsems = (dsem, dsem2)
    esems = (es0, es1)
    wrs01 = (wr0, wr1)
    wsems01 = (ws0, ws1)

    def fetcha(b, p):
        off = base + b * B
        pltpu.sync_copy(dst_h.at[pl.ds(off, B)], didxc.at[b])
        pltpu.async_copy(den_sh.at[didxc.at[b]], denrs[p], dsems[p])
        pltpu.async_copy(ex_h.at[pl.ds(off, B)], exrs[p], esems[p])
        pltpu.sync_copy(src_h.at[pl.ds(off, B)], sidxc.at[b])

    fetcha(0, 0)

    def grpa(g, _):
        for p in range(2):
            b = g * 2 + p
            pltpu.make_async_copy(den_sh.at[didxc.at[0]], denrs[p],
                                  dsems[p]).wait()
            pltpu.make_async_copy(ex_h.at[pl.ds(0, B)], exrs[p],
                                  esems[p]).wait()

            @pl.when(b + 1 < NBAT)
            def _():
                fetcha(b + 1, 1 - p)

            @pl.when(b >= 2)
            def _():
                pltpu.make_async_copy(wrs01[p], w_h.at[pl.ds(0, B)],
                                      wsems01[p]).wait()

            def edge(e, _):
                wrs01[p][e] = exrs[p][e] / denrs[p][e]
                return ()
            lax.fori_loop(0, B, edge, (), unroll=4)
            pltpu.async_copy(wrs01[p], w_h.at[pl.ds(base + b * B, B)],
                             wsems01[p])
        return ()

    lax.fori_loop(0, NBAT // 2, grpa, ())
    for p in range(2):
        pltpu.make_async_copy(wrs01[p], w_h.at[pl.ds(0, B)],
                              wsems01[p]).wait()

    for chunk in range(NCHUNK):
        plsc.subcore_barrier()
        _zero_rows(rows0, CH_W)
        for r in range(NROWS_T // B):
            pltpu.sync_copy(
                rows0, acc_sh.at[pl.ds(sid * NROWS_T + r * B, B)])
        plsc.subcore_barrier()

        h1c = h1cs[chunk]

        def fetch(b, p):
            pltpu.async_copy(h1c.at[sidxc.at[b]], rows[p], gsems[p])
            pltpu.async_copy(w_h.at[pl.ds(base + b * B, B)], wrs[p],
                             wsems[p])

        for p in range(NBUF - 1):
            fetch(p, p)

        def group(g, _):
            for p in range(NBUF):
                b = g * NBUF + p
                pltpu.make_async_copy(h1c.at[sidxc.at[b]], rows[p],
                                      gsems[p]).wait()
                pltpu.make_async_copy(w_h.at[pl.ds(0, B)], wrs[p],
                                      wsems[p]).wait()

                def edge(e, _):
                    w = wrs[p][e][chunk]
                    for j in range(CH_W // 16):
                        rows[p][e, pl.ds(j * 16, 16)] = (
                            rows[p][e, pl.ds(j * 16, 16)] * w)
                    return ()
                lax.fori_loop(0, B, edge, (), unroll=4)

                @pl.when(g < NGRP - 1)
                def _():
                    pltpu.async_copy(rows[p], acc_sh.at[didxc.at[b]],
                                     ssems[p], add=True)

                @pl.when(g == NGRP - 1)
                def _():
                    pltpu.sync_copy(rows[p], acc_sh.at[didxc.at[b]],
                                    add=True)

                pnext = (p + NBUF - 1) % NBUF

                @pl.when(b + NBUF - 1 < NBAT)
                def _():
                    @pl.when(b >= 1)
                    def _():
                        pltpu.make_async_copy(
                            rows[pnext], acc_sh.at[didxc.at[0]],
                            ssems[pnext]).wait()
                    fetch(b + NBUF - 1, pnext)
            return ()

        lax.fori_loop(0, NGRP, group, ())
        plsc.subcore_barrier()
        pltpu.sync_copy(
            acc_sh.at[pl.ds(sid * NROWS_T, NROWS_T)],
            outp_h.at[cid].at[chunk].at[pl.ds(sid * NROWS_T, NROWS_T)])


def _k3(src, dst, ex, denp, h1cs):
    mesh = plsc.VectorSubcoreMesh(core_axis_name="c", subcore_axis_name="s")
    f = pl.kernel(
        _k3_body,
        out_type=[
            jax.ShapeDtypeStruct((2, NCHUNK, N_PAD, CH_W), _f32),
            jax.ShapeDtypeStruct((E_PAD, 16), _f32),
        ],
        mesh=mesh,
        compiler_params=_SC_PARAMS,
        scratch_types=(
            [pltpu.VMEM((NBAT, B), _i32)] * 2
            + [pltpu.VMEM((B, 16), _f32)] * 4
            + [pltpu.VMEM((B, 16), _f32)] * 3
            + [pltpu.VMEM((B, CH_W), _f32)] * 3
            + [pltpu.VMEM_SHARED((N_PAD, CH_W), _f32),
               pltpu.VMEM_SHARED((N_PAD, 16), _f32)]
            + [pltpu.SemaphoreType.DMA] * 13
        ),
    )
    return f(src, dst, ex, denp, *h1cs)


def _k5_body(src_h, dst_h, a2s_h, a2d_h, h2_h, out2p_h,
             a2sl, a2dl, sidx0, sidx1, didx0, didx1, ex2b, den2l, rows, zb1,
             den2_sh, acc2_sh, sem, is0, is1):
    cid = lax.axis_index("c")
    sid = lax.axis_index("s")
    sidxs = (sidx0, sidx1)
    didxs = (didx0, didx1)
    isems = (is0, is1)
    pltpu.sync_copy(a2s_h, a2sl)
    pltpu.sync_copy(a2d_h, a2dl)

    def zrow1(e, _):
        zb1[pl.ds(e * 16, 16)] = jnp.zeros((16,), _f32)
        return ()
    lax.fori_loop(0, NROWS_T // 16, zrow1, ())
    pltpu.sync_copy(zb1, den2_sh.at[pl.ds(sid * NROWS_T, NROWS_T)])
    _zero_rows(rows, OUT2)
    for r in range(NROWS_T // B5):
        pltpu.sync_copy(rows, acc2_sh.at[pl.ds(sid * NROWS_T + r * B5, B5)])
    plsc.subcore_barrier()

    def fetch_idx(base, b, p):
        off = base + b * B5
        pltpu.async_copy(src_h.at[pl.ds(off, B5)], sidxs[p], isems[p])
        pltpu.async_copy(dst_h.at[pl.ds(off, B5)], didxs[p], isems[p])

    def wait_idx(p):
        pltpu.make_async_copy(src_h.at[pl.ds(0, B5)], sidxs[p],
                              isems[p]).wait()
        pltpu.make_async_copy(dst_h.at[pl.ds(0, B5)], didxs[p],
                              isems[p]).wait()

    base1 = sid * ET
    fetch_idx(base1, 0, 0)

    def group1(g, _):
        for p in range(2):
            b = g * 2 + p
            wait_idx(p)

            @pl.when(b + 1 < NBAT51)
            def _():
                fetch_idx(base1, b + 1, 1 - p)
            for gg in range(B5 // 16):
                s16 = sidxs[p][pl.ds(gg * 16, 16)]
                d16 = didxs[p][pl.ds(gg * 16, 16)]
                av = plsc.load_gather(a2sl, [s16])
                dv = plsc.load_gather(a2dl, [d16])
                al = av + dv
                al = jnp.where(al >= 0, al, 0.2 * al)
                ex2b[pl.ds(gg * 16, 16)] = jnp.exp(al)
            pltpu.sync_copy(ex2b, den2_sh.at[didxs[p]], add=True)
        return ()

    lax.fori_loop(0, NBAT51 // 2, group1, ())
    plsc.subcore_barrier()
    pltpu.sync_copy(den2_sh, den2l)

    base2 = sid * ET + cid * EB
    fetch_idx(base2, 0, 0)

    def group2(g, _):
        for p in range(2):
            b = g * 2 + p
            wait_idx(p)

            @pl.when(b + 1 < NBAT5)
            def _():
                fetch_idx(base2, b + 1, 1 - p)
            pltpu.async_copy(h2_h.at[sidxs[p]], rows, sem).wait()

            def grp(gg, _):
                s16 = sidxs[p][pl.ds(gg * 16, 16)]
                d16 = didxs[p][pl.ds(gg * 16, 16)]
                av = plsc.load_gather(a2sl, [s16])
                dv = plsc.load_gather(a2dl, [d16])
                al = av + dv
                al = jnp.where(al >= 0, al, 0.2 * al)
                e16 = jnp.exp(al)
                den16 = plsc.load_gather(den2l, [d16]) + 1e-16
                w16 = e16 / den16
                for l in range(16):
                    w = w16[l]
                    r = gg * 16 + l
                    for j in range(OUT2 // 16):
                        rows[r, pl.ds(j * 16, 16)] = (
                            rows[r, pl.ds(j * 16, 16)] * w)
                return ()
            lax.fori_loop(0, B5 // 16, grp, ())
            pltpu.sync_copy(rows, acc2_sh.at[didxs[p]], add=True)
        return ()

    lax.fori_loop(0, NBAT5 // 2, group2, ())
    plsc.subcore_barrier()
    pltpu.sync_copy(acc2_sh.at[pl.ds(sid * NROWS_T, NROWS_T)],
                    out2p_h.at[cid].at[pl.ds(sid * NROWS_T, NROWS_T)])


def _k5(src, dst, a2s, a2d, h2):
    mesh = plsc.VectorSubcoreMesh(core_axis_name="c", subcore_axis_name="s")
    f = pl.kernel(
        _k5_body,
        out_type=jax.ShapeDtypeStruct((2, N_PAD, OUT2), _f32),
        mesh=mesh,
        compiler_params=_SC_PARAMS,
        scratch_types=[
            pltpu.VMEM((N_PAD,), _f32),
            pltpu.VMEM((N_PAD,), _f32),
            pltpu.VMEM((B5,), _i32),
            pltpu.VMEM((B5,), _i32),
            pltpu.VMEM((B5,), _i32),
            pltpu.VMEM((B5,), _i32),
            pltpu.VMEM((B5,), _f32),
            pltpu.VMEM((N_PAD,), _f32),
            pltpu.VMEM((B5, OUT2), _f32),
            pltpu.VMEM((NROWS_T,), _f32),
            pltpu.VMEM_SHARED((N_PAD,), _f32),
            pltpu.VMEM_SHARED((N_PAD, OUT2), _f32),
            pltpu.SemaphoreType.DMA,
            pltpu.SemaphoreType.DMA,
            pltpu.SemaphoreType.DMA,
        ],
    )
    return f(src, dst, a2s, a2d, h2)


def _prep(x, edge_index, batch, W1, att_src1, att_dst1, b1, W2,
          att_src2, att_dst2):
    xp = jnp.zeros((N_PAD, 80), _f32).at[:N_NODES, :D_IN].set(x)
    w1h = W1.reshape(D_IN, HEADS1, OUT1)
    w1p = jnp.zeros((80, HEADS1, CPAD), _f32).at[:D_IN, :, :OUT1].set(w1h)
    w1p = w1p.reshape(80, HEADS1 * CPAD)
    a_src = jnp.zeros((HEADS1, CPAD, 16), _f32)
    a_dst = jnp.zeros((HEADS1, CPAD, 16), _f32)
    for h in range(HEADS1):
        a_src = a_src.at[h, :OUT1, h].set(att_src1[h])
        a_dst = a_dst.at[h, :OUT1, h].set(att_dst1[h])
    msrc = w1p @ a_src.reshape(HEADS1 * CPAD, 16)
    mdst = w1p @ a_dst.reshape(HEADS1 * CPAD, 16)
    w2h = W2.reshape(HEADS1, OUT1, OUT2)
    w2p = jnp.zeros((HEADS1, CPAD, OUT2), _f32).at[:, :OUT1, :].set(w2h)
    b1p = jnp.zeros((HEADS1, CPAD), _f32).at[:, :OUT1].set(
        b1.reshape(HEADS1, OUT1))
    att2cat = jnp.zeros((OUT2, 8), _f32)
    att2cat = att2cat.at[:, 0].set(att_src2[0]).at[:, 1].set(att_dst2[0])
    loop = jnp.arange(N_NODES, dtype=_i32)
    padv = N_NODES + (jnp.arange(E_PAD - E_TOT, dtype=_i32)
                      % (N_PAD - N_NODES))
    src = jnp.concatenate([edge_index[0], loop, padv])
    dst = jnp.concatenate([edge_index[1], loop, padv])
    batchp = jnp.concatenate(
        [batch, jnp.full((N_PAD - N_NODES,), N_GRAPHS, _i32)])
    return (xp, w1p, msrc, mdst, w2p, b1p, att2cat, src, dst,
            batchp.reshape(N_PAD // 128, 1, 128))


def kernel(x, edge_index, batch, W1, att_src1, att_dst1, b1, W2, att_src2,
           att_dst2, b2, fc_w, fc_b):
    (xp, w1p, msrc, mdst, w2p, b1p, att2cat, src, dst, batchp) = _prep(
        x, edge_index, batch, W1, att_src1, att_dst1, b1, W2,
        att_src2, att_dst2)
    asrc_t, adst_t = _k1a(xp, msrc, mdst)
    h1cs = _k1(xp, w1p)
    ex, denp = _k2(src, dst, asrc_t, adst_t)
    outp, _ = _k3(src, dst, ex, denp, h1cs)
    h2, a2t = _k4(outp, w2p, b1p, att2cat)
    out2p = _k5(src, dst, a2t[:, 0], a2t[:, 1], h2)
    _, res = _k6(out2p, b2.reshape(1, OUT2), batchp, fc_w,
                 fc_b.reshape(1, OUT2))
    return res
